# Initial kernel scaffold; baseline (speedup 1.0000x reference)
import jax
import jax.numpy as jnp
from jax.experimental import pallas as pl


def kernel(h, edge_index, e, emb_h, gtp, merg, layers, mlp):
    raise NotImplementedError("write your pallas kernel here")


# speedup vs baseline: 3.8997x; 3.8997x over previous
import functools

import jax
import jax.numpy as jnp
from jax import lax
from jax.experimental import pallas as pl
from jax.experimental.pallas import tpu as pltpu
from jax.experimental.pallas import tpu_sc as plsc

F32 = jnp.float32
EPS = 1e-5

N = 10000
E = 320000
D = 64
V = 128
NW = 32
N_PAD = 10240
E_PAD = 327680
EPW = E_PAD // NW
NCH = 80
CH = 128
NPW = N_PAD // NW
TPS = N_PAD // 16
DUMMY = N

_SC_MESH = dict(core_axis_name="c", subcore_axis_name="s")

CHL = 32
NRB = 4
CPB = NRB * (CH // CHL)
NBLKL = NCH // NRB
NCHL = EPW // CHL
E_HALF = E_PAD // 2


def _wid():
    return lax.axis_index("c") * 16 + lax.axis_index("s")


def _sc0_body(h_hbm, src_hbm, dst_hbm, nt_hbm,
              pid_o, hs_o, hd_o, ntg_o,
              hv, sv, dv, pv, hsv, hdv, bufs, bufd, bufn, sema, semb, sem):
    wid = _wid()
    pltpu.sync_copy(h_hbm, hv)
    pltpu.sync_copy(src_hbm.at[wid], sv)
    pltpu.sync_copy(dst_hbm.at[wid], dv)

    def chunk(j, _):
        ga = pltpu.async_copy(h_hbm.at[sv.at[j]], bufs, sema)
        gb = pltpu.async_copy(h_hbm.at[dv.at[j]], bufd, semb)
        ga.wait()
        gb.wait()

        def step(k, _2):
            sl = pl.ds(k * 16, 16)
            hs = bufs[sl]
            hd = bufd[sl]
            o = pl.ds(j * CH + k * 16, 16)
            pv[o] = hs * V + hd
            hsv[o] = hs
            hdv[o] = hd
            return 0

        lax.fori_loop(0, CH // 16, step, 0)
        return 0

    lax.fori_loop(0, NCH, chunk, 0)
    pltpu.sync_copy(pv, pid_o.at[wid])
    pltpu.sync_copy(hsv, hs_o.at[wid])
    pltpu.sync_copy(hdv, hd_o.at[wid])

    nb = wid * NPW
    for off, ln in ((0, 128), (128, 128), (256, 64)):
        pltpu.async_copy(nt_hbm.at[hv.at[pl.ds(nb + off, ln)]],
                         bufn.at[pl.ds(0, ln)], sem).wait()
        pltpu.sync_copy(bufn.at[pl.ds(0, ln)],
                        ntg_o.at[pl.ds(nb + off, ln), :])


def _sc0(h_pad, src3, dst3, nt):
    fn = pl.kernel(
        _sc0_body,
        out_type=(
            jax.ShapeDtypeStruct((NW, EPW), jnp.int32),
            jax.ShapeDtypeStruct((NW, EPW), jnp.int32),
            jax.ShapeDtypeStruct((NW, EPW), jnp.int32),
            jax.ShapeDtypeStruct((N_PAD, 2 * D), F32),
        ),
        mesh=plsc.VectorSubcoreMesh(**_SC_MESH),
        scratch_types=(
            pltpu.VMEM((N_PAD,), jnp.int32),
            pltpu.VMEM((NCH, CH), jnp.int32),
            pltpu.VMEM((NCH, CH), jnp.int32),
            pltpu.VMEM((EPW,), jnp.int32),
            pltpu.VMEM((EPW,), jnp.int32),
            pltpu.VMEM((EPW,), jnp.int32),
            pltpu.VMEM((CH,), jnp.int32),
            pltpu.VMEM((CH,), jnp.int32),
            pltpu.VMEM((128, 2 * D), F32),
            pltpu.SemaphoreType.DMA,
            pltpu.SemaphoreType.DMA,
            pltpu.SemaphoreType.DMA,
        ),
    )
    return fn(h_pad, src3, dst3, nt)


def _sc_l1_body(pid_hbm, dst_hbm, tsg_hbm, z_hbm,
                dn_o,
                pv, dv, buf0, buf1, dn_s, sem0, sem1):
    cid = lax.axis_index("c")
    sid = lax.axis_index("s")
    wid = cid * 16 + sid
    tile_rows = pl.ds(sid * TPS, TPS)
    pltpu.sync_copy(z_hbm, dn_s.at[tile_rows, :])
    plsc.subcore_barrier()

    def blk(b, _):
        pltpu.sync_copy(pid_hbm.at[wid, pl.ds(b * 8, 8)], pv)
        pltpu.sync_copy(dst_hbm.at[wid, pl.ds(b * 8, 8)], dv)

        def pair(g, _2):
            ia = pv.at[g, pl.ds(0, 64)]
            ib = pv.at[g, pl.ds(64, 64)]
            ga = pltpu.async_copy(tsg_hbm.at[ia], buf0, sem0)
            gb = pltpu.async_copy(tsg_hbm.at[ib], buf1, sem1)
            ga.wait()
            pltpu.sync_copy(buf0, dn_s.at[dv.at[g, pl.ds(0, 64)]], add=True)
            gb.wait()
            pltpu.sync_copy(buf1, dn_s.at[dv.at[g, pl.ds(64, 64)]], add=True)
            return 0

        lax.fori_loop(0, 8, pair, 0)
        return 0

    lax.fori_loop(0, NCH // 8, blk, 0)
    plsc.subcore_barrier()
    pltpu.sync_copy(dn_s.at[tile_rows, :], dn_o.at[cid, tile_rows, :])


def _sc_l1(pid3, dst3, tsg, zrow):
    fn = pl.kernel(
        _sc_l1_body,
        out_type=(
            jax.ShapeDtypeStruct((2, N_PAD, 2 * D), F32),
        ),
        mesh=plsc.VectorSubcoreMesh(**_SC_MESH),
        scratch_types=(
            pltpu.VMEM((8, CH), jnp.int32),
            pltpu.VMEM((8, CH), jnp.int32),
            pltpu.VMEM((64, 2 * D), F32),
            pltpu.VMEM((64, 2 * D), F32),
            pltpu.VMEM_SHARED((N_PAD, 2 * D), F32),
            pltpu.SemaphoreType.DMA,
            pltpu.SemaphoreType.DMA,
        ),
    )
    return fn(pid3, dst3, tsg, zrow)[0]


def _make_sc_layer(gather_ce, with_stats):
    def body(*refs):
        it = iter(refs)
        src_hbm = next(it)
        dst_hbm = next(it)
        if gather_ce:
            pid_hbm = next(it)
            tq_hbm = next(it)
        else:
            ce_hbm = next(it)
        dh_hbm = next(it)
        eb_hbm = next(it)
        z_hbm = next(it)
        dn_o = next(it)
        if with_stats:
            ehat_o = next(it)
            ss_o = next(it)
        if gather_ce:
            eein_o = next(it)
        sv = next(it)
        dv = next(it)
        if gather_ce:
            pv = next(it)
            bufi = next(it)
        else:
            bufc = next(it)
        bufd = next(it)
        bufeb = next(it)
        bufo = next(it)
        bufe = next(it)
        if gather_ce:
            bufw = next(it)
        bufs = next(it)
        dn_s = next(it)
        sem0 = next(it)
        sem1 = next(it)
        sem2 = next(it)

        cid = lax.axis_index("c")
        sid = lax.axis_index("s")
        wid = cid * 16 + sid
        base = wid * EPW
        tile_rows = pl.ds(sid * TPS, TPS)
        pltpu.sync_copy(z_hbm, dn_s.at[tile_rows, :])
        plsc.subcore_barrier()
        nreal = jnp.where(wid == NW - 1, (E - (NW - 1) * EPW) // CHL, NCHL)

        zv = jnp.zeros((16,), F32)

        def blk(b, bcarry):
            pltpu.sync_copy(src_hbm.at[wid, pl.ds(b * NRB, NRB)], sv)
            pltpu.sync_copy(dst_hbm.at[wid, pl.ds(b * NRB, NRB)], dv)
            if gather_ce:
                pltpu.sync_copy(pid_hbm.at[wid, pl.ds(b * NRB, NRB)], pv)

            def chunk(g, carry):
                row = g // 4
                col = (g % 4) * CHL
                di = dv.at[row, pl.ds(col, CHL)]
                si = sv.at[row, pl.ds(col, CHL)]
                gd = pltpu.async_copy(dh_hbm.at[di], bufd, sem0)
                ge = pltpu.async_copy(eb_hbm.at[si], bufeb, sem1)
                off = base + (b * CPB + g) * CHL
                hoff = pl.multiple_of(off // 2, CHL // 2)
                if gather_ce:
                    pi = pv.at[row, pl.ds(col, CHL)]
                    gc = pltpu.async_copy(tq_hbm.at[pi], bufi, sem2)
                else:
                    gc = pltpu.async_copy(
                        ce_hbm.at[pl.ds(hoff, CHL // 2), :], bufc, sem2)
                gd.wait()
                ge.wait()
                gc.wait()

                def prow(p, rc):
                    out = list(rc)
                    for edge in range(2):
                        r = 2 * p + edge
                        for c in range(4):
                            sl = pl.ds(c * 16, 16)
                            sl2 = pl.ds(D + c * 16, 16)
                            pcol = pl.ds(edge * D + c * 16, 16)
                            if gather_ce:
                                ce_v = bufi[r, sl]
                            else:
                                ce_v = bufc[p, pcol]
                            eh = bufd[r, sl] + bufeb[r, sl] + ce_v
                            bh = bufeb[r, sl2]
                            sg = 1.0 / (1.0 + jnp.exp(-eh))
                            bufo[r, sl] = sg
                            bufo[r, sl2] = sg * bh
                            if with_stats:
                                bufe[p, pcol] = eh
                                out[c] = out[c] + eh
                                out[4 + c] = out[4 + c] + eh * eh
                            if gather_ce:
                                bufw[p, pcol] = bufi[r, sl2]
                    return tuple(out)

                csum = lax.fori_loop(0, CHL // 2, prow, (zv,) * 8)
                pltpu.sync_copy(bufo, dn_s.at[di], add=True)
                if with_stats:
                    pltpu.sync_copy(bufe,
                                    ehat_o.at[pl.ds(hoff, CHL // 2), :])
                if gather_ce:
                    pltpu.sync_copy(bufw,
                                    eein_o.at[pl.ds(hoff, CHL // 2), :])
                m = jnp.where(b * CPB + g < nreal, 1.0, 0.0).astype(F32)
                return tuple(carry[i] + m * csum[i] for i in range(8))

            return lax.fori_loop(0, CPB, chunk, bcarry)

        stats = lax.fori_loop(0, NBLKL, blk, (zv,) * 8)
        if with_stats:
            for c in range(4):
                bufs[pl.ds(c * 16, 16)] = stats[c]
                bufs[pl.ds(D + c * 16, 16)] = stats[4 + c]
            pltpu.sync_copy(bufs, ss_o.at[wid])
        plsc.subcore_barrier()
        pltpu.sync_copy(dn_s.at[tile_rows, :], dn_o.at[cid, tile_rows, :])

    out_type = [
        jax.ShapeDtypeStruct((2, N_PAD, 2 * D), F32),
    ]
    if with_stats:
        out_type += [
            jax.ShapeDtypeStruct((E_HALF, 2 * D), F32),
            jax.ShapeDtypeStruct((NW, 2 * D), F32),
        ]
    if gather_ce:
        out_type += [jax.ShapeDtypeStruct((E_HALF, 2 * D), F32)]

    scratch = [
        pltpu.VMEM((NRB, CH), jnp.int32),
        pltpu.VMEM((NRB, CH), jnp.int32),
    ]
    if gather_ce:
        scratch += [pltpu.VMEM((NRB, CH), jnp.int32),
                    pltpu.VMEM((CHL, 2 * D), F32)]
    else:
        scratch += [pltpu.VMEM((CHL // 2, 2 * D), F32)]
    scratch += [
        pltpu.VMEM((CHL, 2 * D), F32),
        pltpu.VMEM((CHL, 2 * D), F32),
        pltpu.VMEM((CHL, 2 * D), F32),
        pltpu.VMEM((CHL // 2, 2 * D), F32),
    ]
    if gather_ce:
        scratch += [pltpu.VMEM((CHL // 2, 2 * D), F32)]
    scratch += [
        pltpu.VMEM((2 * D,), F32),
        pltpu.VMEM_SHARED((N_PAD, 2 * D), F32),
        pltpu.SemaphoreType.DMA,
        pltpu.SemaphoreType.DMA,
        pltpu.SemaphoreType.DMA,
    ]
    return pl.kernel(
        body, out_type=tuple(out_type),
        mesh=plsc.VectorSubcoreMesh(**_SC_MESH),
        scratch_types=tuple(scratch))


def _dot(a, b):
    return jnp.dot(a, b, preferred_element_type=F32)


def _tab0_body(emb_ref, w1_ref, b1_ref, w2_ref, b2_ref, wetop_ref, webot_ref,
               se_ref, de_ref, a_ref, ab_ref, b_ref, bb_ref, d_ref, db_ref,
               e_ref, eb_ref,
               p1_o, p2_o, l1_o, l2_o, tb_o, td_o, te_o, nt_o):
    emb = emb_ref[...]
    p1_o[...] = _dot(emb, w1_ref[...]) + b1_ref[...]
    p2_o[...] = _dot(emb, w2_ref[...]) + b2_ref[...]
    l1_o[...] = _dot(se_ref[...], wetop_ref[...])
    l2_o[...] = _dot(de_ref[...], webot_ref[...])
    tb_o[...] = _dot(emb, b_ref[...]) + bb_ref[...]
    td_o[...] = _dot(emb, d_ref[...]) + db_ref[...]
    te_o[...] = _dot(emb, e_ref[...]) + eb_ref[...]
    ta = _dot(emb, a_ref[...]) + ab_ref[...]
    nt_o[...] = jnp.concatenate([emb, ta], axis=1)


def _tab0(emb_h, mp, lp0):
    full = pl.BlockSpec(None, lambda: tuple())
    outs = (
        jax.ShapeDtypeStruct((V, D * D), F32),
        jax.ShapeDtypeStruct((V, D), F32),
        jax.ShapeDtypeStruct((V, D), F32),
        jax.ShapeDtypeStruct((V, D), F32),
        jax.ShapeDtypeStruct((V, D), F32),
        jax.ShapeDtypeStruct((V, D), F32),
        jax.ShapeDtypeStruct((V, D), F32),
        jax.ShapeDtypeStruct((V, 2 * D), F32),
    )
    args = (emb_h, mp['W1'], mp['b1'].reshape(1, -1), mp['W2'],
            mp['b2'].reshape(1, -1), mp['W_ep'][:D], mp['W_ep'][D:],
            mp['src_emb'], mp['dst_emb'],
            lp0['A'], lp0['Ab'].reshape(1, -1), lp0['B'],
            lp0['Bb'].reshape(1, -1), lp0['D'], lp0['Db'].reshape(1, -1),
            lp0['E'], lp0['Eb'].reshape(1, -1))
    return pl.pallas_call(
        _tab0_body,
        out_shape=outs,
    )(*args)


_CB = 2000


def _count_body(hs_ref, hd_ref, cnt_o, acc):
    i = pl.program_id(0)

    @pl.when(i == 0)
    def _():
        acc[...] = jnp.zeros_like(acc)

    lanes = lax.broadcasted_iota(jnp.int32, (_CB, V), 1)
    ohs = (hs_ref[...] == lanes).astype(jnp.bfloat16)
    ohd = (hd_ref[...] == lanes).astype(jnp.bfloat16)
    acc[...] += lax.dot_general(ohs, ohd, (((0,), (0,)), ((), ())),
                                preferred_element_type=F32)

    @pl.when(i == pl.num_programs(0) - 1)
    def _():
        cnt_o[...] = acc[...]


def _count(hs_col, hd_col):
    grid = E // _CB
    return pl.pallas_call(
        _count_body,
        grid=(grid,),
        in_specs=[pl.BlockSpec((_CB, 1), lambda i: (i, 0)),
                  pl.BlockSpec((_CB, 1), lambda i: (i, 0))],
        out_specs=pl.BlockSpec((V, V), lambda i: (0, 0)),
        out_shape=jax.ShapeDtypeStruct((V, V), F32),
        scratch_shapes=[pltpu.VMEM((V, V), F32)],
    )(hs_col, hd_col)


def _tab1_body(p1_ref, p2_ref, w3_ref, b3_ref, l1_ref, l2_ref, bep_ref,
               cnt_ref, tx_o, s1_o, s2_o, s1a, s2a):
    u = pl.program_id(0)

    @pl.when(u == 0)
    def _():
        s1a[...] = jnp.zeros_like(s1a)
        s2a[...] = jnp.zeros_like(s2a)

    p1u = p1_ref[0]
    tu = lax.dot_general(p2_ref[...], p1u, (((1,), (1,)), ((), ())),
                         preferred_element_type=F32)
    tx = (_dot(tu, w3_ref[...]) + b3_ref[...] + l1_ref[0]
          + l2_ref[...] + bep_ref[...])
    tx_o[...] = tx
    w = cnt_ref[0] * (1.0 / E)
    s1a[...] += _dot(w, tx)
    s2a[...] += _dot(w, tx * tx)

    @pl.when(u == pl.num_programs(0) - 1)
    def _():
        s1_o[...] = s1a[...]
        s2_o[...] = s2a[...]


def _tab1(p1v3, p2v, mp, l1t, l2t, cnt):
    return pl.pallas_call(
        _tab1_body,
        grid=(V,),
        in_specs=[
            pl.BlockSpec((1, D, D), lambda u: (u, 0, 0)),
            pl.BlockSpec((V, D), lambda u: (0, 0)),
            pl.BlockSpec((D, D), lambda u: (0, 0)),
            pl.BlockSpec((1, D), lambda u: (0, 0)),
            pl.BlockSpec((1, 1, D), lambda u: (u, 0, 0)),
            pl.BlockSpec((V, D), lambda u: (0, 0)),
            pl.BlockSpec((1, D), lambda u: (0, 0)),
            pl.BlockSpec((1, 1, V), lambda u: (u, 0, 0)),
        ],
        out_specs=[
            pl.BlockSpec((V, D), lambda u: (u, 0)),
            pl.BlockSpec((1, D), lambda u: (0, 0)),
            pl.BlockSpec((1, D), lambda u: (0, 0)),
        ],
        out_shape=[
            jax.ShapeDtypeStruct((V * V, D), F32),
            jax.ShapeDtypeStruct((1, D), F32),
            jax.ShapeDtypeStruct((1, D), F32),
        ],
        scratch_shapes=[pltpu.VMEM((1, D), F32), pltpu.VMEM((1, D), F32)],
    )(p1v3, p2v, mp['W_ep3'], mp['b_ep3'].reshape(1, -1),
      l1t.reshape(V, 1, D), l2t, mp['b_ep'].reshape(1, -1),
      cnt.reshape(V, 1, V))


def _tab2_body(tx_ref, s1_ref, s2_ref, g_ref, b_ref, td_ref, te_ref, tb_ref,
               c_ref, cb_ref, cnt_ref,
               tee_o, teh_o, tsg_o, s1e_o, s2e_o, s1a, s2a):
    u = pl.program_id(0)

    @pl.when(u == 0)
    def _():
        s1a[...] = jnp.zeros_like(s1a)
        s2a[...] = jnp.zeros_like(s2a)

    mu = s1_ref[...]
    inv = lax.rsqrt(s2_ref[...] - mu * mu + EPS)
    tee = jnp.maximum((tx_ref[...] - mu) * inv * g_ref[...] + b_ref[...], 0.0)
    tee_o[...] = tee
    teh = _dot(tee, c_ref[...]) + cb_ref[...] + td_ref[...] + te_ref[0]
    teh_o[...] = teh
    tsig = jax.nn.sigmoid(teh)
    tsg_o[...] = jnp.concatenate([tsig, tsig * tb_ref[0]], axis=1)
    w = cnt_ref[0] * (1.0 / E)
    s1a[...] += _dot(w, teh)
    s2a[...] += _dot(w, teh * teh)

    @pl.when(u == pl.num_programs(0) - 1)
    def _():
        s1e_o[...] = s1a[...]
        s2e_o[...] = s2a[...]


def _tab2(tx, s1m, s2m, mp, lp0, tbt, tdt, tet, cnt):
    return pl.pallas_call(
        _tab2_body,
        grid=(V,),
        in_specs=[
            pl.BlockSpec((V, D), lambda u: (u, 0)),
            pl.BlockSpec((1, D), lambda u: (0, 0)),
            pl.BlockSpec((1, D), lambda u: (0, 0)),
            pl.BlockSpec((1, D), lambda u: (0, 0)),
            pl.BlockSpec((1, D), lambda u: (0, 0)),
            pl.BlockSpec((V, D), lambda u: (0, 0)),
            pl.BlockSpec((1, 1, D), lambda u: (u, 0, 0)),
            pl.BlockSpec((1, 1, D), lambda u: (u, 0, 0)),
            pl.BlockSpec((D, D), lambda u: (0, 0)),
            pl.BlockSpec((1, D), lambda u: (0, 0)),
            pl.BlockSpec((1, 1, V), lambda u: (u, 0, 0)),
        ],
        out_specs=[
            pl.BlockSpec((V, D), lambda u: (u, 0)),
            pl.BlockSpec((V, D), lambda u: (u, 0)),
            pl.BlockSpec((V, 2 * D), lambda u: (u, 0)),
            pl.BlockSpec((1, D), lambda u: (0, 0)),
            pl.BlockSpec((1, D), lambda u: (0, 0)),
        ],
        out_shape=[
            jax.ShapeDtypeStruct((V * V, D), F32),
            jax.ShapeDtypeStruct((V * V, D), F32),
            jax.ShapeDtypeStruct((V * V, 2 * D), F32),
            jax.ShapeDtypeStruct((1, D), F32),
            jax.ShapeDtypeStruct((1, D), F32),
        ],
        scratch_shapes=[pltpu.VMEM((1, D), F32), pltpu.VMEM((1, D), F32)],
    )(tx, s1m, s2m, mp['bn_g'].reshape(1, -1), mp['bn_b'].reshape(1, -1),
      tdt, tet.reshape(V, 1, D), tbt.reshape(V, 1, D), lp0['C'],
      lp0['Cb'].reshape(1, -1), cnt.reshape(V, 1, V))


def _tab3_body(tee_ref, teh_ref, s1_ref, s2_ref, g_ref, b_ref, c_ref, cb_ref,
               tq_o):
    mu = s1_ref[...]
    inv = lax.rsqrt(s2_ref[...] - mu * mu + EPS)
    te2 = tee_ref[...] + jnp.maximum(
        (teh_ref[...] - mu) * inv * g_ref[...] + b_ref[...], 0.0)
    tq_o[...] = jnp.concatenate(
        [_dot(te2, c_ref[...]) + cb_ref[...], te2], axis=1)


def _tab3(tee, teh, s1e, s2e, lp0, lp1):
    return pl.pallas_call(
        _tab3_body,
        grid=(V,),
        in_specs=[
            pl.BlockSpec((V, D), lambda u: (u, 0)),
            pl.BlockSpec((V, D), lambda u: (u, 0)),
            pl.BlockSpec((1, D), lambda u: (0, 0)),
            pl.BlockSpec((1, D), lambda u: (0, 0)),
            pl.BlockSpec((1, D), lambda u: (0, 0)),
            pl.BlockSpec((1, D), lambda u: (0, 0)),
            pl.BlockSpec((D, D), lambda u: (0, 0)),
            pl.BlockSpec((1, D), lambda u: (0, 0)),
        ],
        out_specs=pl.BlockSpec((V, 2 * D), lambda u: (u, 0)),
        out_shape=jax.ShapeDtypeStruct((V * V, 2 * D), F32),
    )(tee, teh, s1e, s2e, lp0['bne_g'].reshape(1, -1),
      lp0['bne_b'].reshape(1, -1), lp1['C'], lp1['Cb'].reshape(1, -1))


_NB = 2048


def _row_mask(nrows, limit):
    rows = (pl.program_id(0) * nrows
            + lax.broadcasted_iota(jnp.int32, (nrows, 1), 0))
    return rows < limit


def _mm_body(hh_ref, a_ref, ab_ref, b_ref, bb_ref, d_ref, db_ref,
             e_ref, eb_ref, ah_o, dh_o, ebo_o):
    m = _row_mask(_NB, N)
    hh = hh_ref[...]
    ah_o[...] = jnp.where(m, _dot(hh, a_ref[...]) + ab_ref[...], 0.0)
    dh = jnp.where(m, _dot(hh, d_ref[...]) + db_ref[...], 0.0)
    dh_o[...] = jnp.concatenate([dh, jnp.zeros_like(dh)], axis=1)
    eh = _dot(hh, e_ref[...]) + eb_ref[...]
    bh = _dot(hh, b_ref[...]) + bb_ref[...]
    ebo_o[...] = jnp.where(m, jnp.concatenate([eh, bh], axis=1), 0.0)


def _tc_mm(hh, lp):
    return pl.pallas_call(
        _mm_body,
        grid=(N_PAD // _NB,),
        in_specs=[pl.BlockSpec((_NB, D), lambda i: (i, 0))]
        + [pl.BlockSpec((D, D), lambda i: (0, 0)),
           pl.BlockSpec((1, D), lambda i: (0, 0))] * 4,
        out_specs=[
            pl.BlockSpec((_NB, D), lambda i: (i, 0)),
            pl.BlockSpec((_NB, 2 * D), lambda i: (i, 0)),
            pl.BlockSpec((_NB, 2 * D), lambda i: (i, 0)),
        ],
        out_shape=[
            jax.ShapeDtypeStruct((N_PAD, D), F32),
            jax.ShapeDtypeStruct((N_PAD, 2 * D), F32),
            jax.ShapeDtypeStruct((N_PAD, 2 * D), F32),
        ],
    )(hh, lp['A'], lp['Ab'].reshape(1, -1), lp['B'], lp['Bb'].reshape(1, -1),
      lp['D'], lp['Db'].reshape(1, -1), lp['E'], lp['Eb'].reshape(1, -1))


def _ha_body(ah_ref, dn_ref, hn_o, s1_o, s2_o, s1a, s2a):
    i = pl.program_id(0)

    @pl.when(i == 0)
    def _():
        s1a[...] = jnp.zeros_like(s1a)
        s2a[...] = jnp.zeros_like(s2a)

    dn = dn_ref[0] + dn_ref[1]
    den = dn[:, :D]
    num = dn[:, D:]
    hn = ah_ref[...] + num / (den + 1e-6)
    hn = jnp.where(_row_mask(_NB, N), hn, 0.0)
    hn_o[...] = hn
    s1a[...] += jnp.sum(hn, axis=0, keepdims=True)
    s2a[...] += jnp.sum(hn * hn, axis=0, keepdims=True)

    @pl.when(i == pl.num_programs(0) - 1)
    def _():
        s1_o[...] = s1a[...]
        s2_o[...] = s2a[...]


def _tc_ha(ah, dn_p):
    return pl.pallas_call(
        _ha_body,
        grid=(N_PAD // _NB,),
        in_specs=[
            pl.BlockSpec((_NB, D), lambda i: (i, 0)),
            pl.BlockSpec((2, _NB, 2 * D), lambda i: (0, i, 0)),
        ],
        out_specs=[
            pl.BlockSpec((_NB, D), lambda i: (i, 0)),
            pl.BlockSpec((1, D), lambda i: (0, 0)),
            pl.BlockSpec((1, D), lambda i: (0, 0)),
        ],
        out_shape=[
            jax.ShapeDtypeStruct((N_PAD, D), F32),
            jax.ShapeDtypeStruct((1, D), F32),
            jax.ShapeDtypeStruct((1, D), F32),
        ],
        scratch_shapes=[pltpu.VMEM((1, D), F32), pltpu.VMEM((1, D), F32)],
    )(ah, dn_p)


def _hb_body(hh_ref, hn_ref, s1_ref, s2_ref, g_ref, b_ref, hho_o):
    mu = s1_ref[...] * (1.0 / N)
    var = s2_ref[...] * (1.0 / N) - mu * mu
    inv = lax.rsqrt(var + EPS)
    hho = hh_ref[...] + jnp.maximum(
        (hn_ref[...] - mu) * inv * g_ref[...] + b_ref[...], 0.0)
    hho_o[...] = jnp.where(_row_mask(_NB, N), hho, 0.0)


def _tc_hb(hh, hn, s1, s2, lp):
    return pl.pallas_call(
        _hb_body,
        grid=(N_PAD // _NB,),
        in_specs=[
            pl.BlockSpec((_NB, D), lambda i: (i, 0)),
            pl.BlockSpec((_NB, D), lambda i: (i, 0)),
            pl.BlockSpec((1, D), lambda i: (0, 0)),
            pl.BlockSpec((1, D), lambda i: (0, 0)),
            pl.BlockSpec((1, D), lambda i: (0, 0)),
            pl.BlockSpec((1, D), lambda i: (0, 0)),
        ],
        out_specs=pl.BlockSpec((_NB, D), lambda i: (i, 0)),
        out_shape=jax.ShapeDtypeStruct((N_PAD, D), F32),
    )(hh, hn, s1, s2, lp['bnh_g'].reshape(1, -1), lp['bnh_b'].reshape(1, -1))


_EB = 2048


def _make_e_body(with_ce):
    def body(*refs):
        if with_ce:
            (eh_ref, ee_ref, ss_ref, g_ref, b_ref, c_ref, cb_ref,
             eo_o, ce_o) = refs
        else:
            (eh_ref, ee_ref, ss_ref, g_ref, b_ref, eo_o) = refs
        ss = jnp.sum(ss_ref[...], axis=0, keepdims=True) * (1.0 / E)
        mu = ss[:, :D]
        inv = lax.rsqrt(ss[:, D:] - mu * mu + EPS)
        mu2 = jnp.concatenate([mu, mu], axis=1)
        inv2 = jnp.concatenate([inv, inv], axis=1)
        g2 = jnp.concatenate([g_ref[...], g_ref[...]], axis=1)
        b2 = jnp.concatenate([b_ref[...], b_ref[...]], axis=1)
        eo = ee_ref[...] + jnp.maximum(
            (eh_ref[...] - mu2) * inv2 * g2 + b2, 0.0)
        m = _row_mask(_EB, E // 2)
        eo = jnp.where(m, eo, 0.0)
        eo_o[...] = eo
        if with_ce:
            ce_o[...] = _dot(eo, c_ref[...]) + cb_ref[...]
    return body


def _tc_e(ehat, eein, ss, lp, lp_next):
    with_ce = lp_next is not None
    in_specs = [
        pl.BlockSpec((_EB, 2 * D), lambda i: (i, 0)),
        pl.BlockSpec((_EB, 2 * D), lambda i: (i, 0)),
        pl.BlockSpec((NW, 2 * D), lambda i: (0, 0)),
        pl.BlockSpec((1, D), lambda i: (0, 0)),
        pl.BlockSpec((1, D), lambda i: (0, 0)),
    ]
    args = [ehat, eein, ss, lp['bne_g'].reshape(1, -1),
            lp['bne_b'].reshape(1, -1)]
    out_specs = [pl.BlockSpec((_EB, 2 * D), lambda i: (i, 0))]
    out_shape = [jax.ShapeDtypeStruct((E_HALF, 2 * D), F32)]
    if with_ce:
        zdd = jnp.zeros((D, D), F32)
        c2 = jnp.block([[lp_next['C'], zdd], [zdd, lp_next['C']]])
        cb2 = jnp.tile(lp_next['Cb'], 2).reshape(1, -1)
        in_specs += [pl.BlockSpec((2 * D, 2 * D), lambda i: (0, 0)),
                     pl.BlockSpec((1, 2 * D), lambda i: (0, 0))]
        args += [c2, cb2]
        out_specs += [pl.BlockSpec((_EB, 2 * D), lambda i: (i, 0))]
        out_shape += [jax.ShapeDtypeStruct((E_HALF, 2 * D), F32)]
    res = pl.pallas_call(
        _make_e_body(with_ce),
        grid=(E_HALF // _EB,),
        in_specs=in_specs,
        out_specs=out_specs,
        out_shape=out_shape,
    )(*args)
    return res if with_ce else (res[0], None)


def _mlp_body(hh_ref, w0_ref, b0_ref, w1_ref, b1_ref, w2_ref, b2_ref, y_o):
    y = jnp.maximum(_dot(hh_ref[...], w0_ref[...]) + b0_ref[...], 0.0)
    y = jnp.maximum(_dot(y, w1_ref[...]) + b1_ref[...], 0.0)
    y_o[...] = _dot(y, w2_ref[...]) + b2_ref[...]


def _tc_mlp(hh, mlp):
    d0 = mlp['W0'].shape[1]
    d1 = mlp['W1'].shape[1]
    d2 = mlp['W2'].shape[1]
    return pl.pallas_call(
        _mlp_body,
        grid=(N_PAD // _NB,),
        in_specs=[
            pl.BlockSpec((_NB, D), lambda i: (i, 0)),
            pl.BlockSpec((D, d0), lambda i: (0, 0)),
            pl.BlockSpec((1, d0), lambda i: (0, 0)),
            pl.BlockSpec((d0, d1), lambda i: (0, 0)),
            pl.BlockSpec((1, d1), lambda i: (0, 0)),
            pl.BlockSpec((d1, d2), lambda i: (0, 0)),
            pl.BlockSpec((1, d2), lambda i: (0, 0)),
        ],
        out_specs=pl.BlockSpec((_NB, d2), lambda i: (i, 0)),
        out_shape=jax.ShapeDtypeStruct((N_PAD, d2), F32),
    )(hh, mlp['W0'], mlp['b0'].reshape(1, -1), mlp['W1'],
      mlp['b1'].reshape(1, -1), mlp['W2'], mlp['b2'].reshape(1, -1))


def kernel(h, edge_index, e, emb_h, gtp, merg, layers, mlp):
    del e, gtp
    h = h.astype(jnp.int32)
    src = edge_index[0].astype(jnp.int32)
    dst = edge_index[1].astype(jnp.int32)

    h_pad = jnp.concatenate([h, jnp.zeros((N_PAD - N,), jnp.int32)])
    pad_e = jnp.full((E_PAD - E,), DUMMY, jnp.int32)
    src2 = jnp.concatenate([src, pad_e]).reshape(NW, EPW)
    dst2 = jnp.concatenate([dst, pad_e]).reshape(NW, EPW)
    zrow = jnp.zeros((TPS, 2 * D), F32)

    lp = layers

    p1v, p2v, l1t, l2t, tbt, tdt, tet, nt = _tab0(emb_h, merg, lp[0])
    src3 = src2.reshape(NW, NCH, CH)
    dst3 = dst2.reshape(NW, NCH, CH)
    pid2, hs2, hd2, ntg = _sc0(h_pad, src3, dst3, nt)
    pid3 = pid2.reshape(NW, NCH, CH)
    hs_col = hs2.reshape(E_PAD, 1)[:E]
    hd_col = hd2.reshape(E_PAD, 1)[:E]

    cnt = _count(hs_col, hd_col)
    tx, s1m, s2m = _tab1(p1v.reshape(V, D, D), p2v, merg, l1t, l2t, cnt)
    tee, teh, tsg, s1e1, s2e1 = _tab2(
        tx, s1m, s2m, merg, lp[0], tbt, tdt, tet, cnt)
    tq = _tab3(tee, teh, s1e1, s2e1, lp[0], lp[1])

    dn_p = _sc_l1(pid3, dst3, tsg, zrow)
    hemb = ntg[:, :D]
    ah1 = ntg[:, D:]
    hn, s1h, s2h = _tc_ha(ah1, dn_p)
    hh = _tc_hb(hemb, hn, s1h, s2h, lp[0])

    sc_l2 = _make_sc_layer(gather_ce=True, with_stats=True)
    sc_l3 = _make_sc_layer(gather_ce=False, with_stats=True)
    sc_l4 = _make_sc_layer(gather_ce=False, with_stats=False)

    ah, dh, ebt = _tc_mm(hh, lp[1])
    dn_p, ehat, ss, eein = sc_l2(src3, dst3, pid3, tq, dh, ebt, zrow)
    hn, s1h, s2h = _tc_ha(ah, dn_p)
    hh = _tc_hb(hh, hn, s1h, s2h, lp[1])
    ee, ce = _tc_e(ehat, eein, ss, lp[1], lp[2])

    ah, dh, ebt = _tc_mm(hh, lp[2])
    dn_p, ehat, ss = sc_l3(src3, dst3, ce, dh, ebt, zrow)
    hn, s1h, s2h = _tc_ha(ah, dn_p)
    hh = _tc_hb(hh, hn, s1h, s2h, lp[2])
    ee, ce = _tc_e(ehat, ee, ss, lp[2], lp[3])

    ah, dh, ebt = _tc_mm(hh, lp[3])
    dn_p, = sc_l4(src3, dst3, ce, dh, ebt, zrow)
    hn, s1h, s2h = _tc_ha(ah, dn_p)
    hh = _tc_hb(hh, hn, s1h, s2h, lp[3])

    return _tc_mlp(hh, mlp)[:N]


# speedup vs baseline: 5.4764x; 1.4043x over previous
import functools

import jax
import jax.numpy as jnp
from jax import lax
from jax.experimental import pallas as pl
from jax.experimental.pallas import tpu as pltpu
from jax.experimental.pallas import tpu_sc as plsc

F32 = jnp.float32
EPS = 1e-5

N = 10000
E = 320000
D = 64
V = 128
NW = 32
N_PAD = 10240
E_PAD = 327680
EPW = E_PAD // NW
NCH = 80
CH = 128
NPW = N_PAD // NW
TPS = N_PAD // 16
DUMMY = N

_SC_MESH = dict(core_axis_name="c", subcore_axis_name="s")

CHL = 32
NRB = 4
CPB = NRB * (CH // CHL)
NBLKL = NCH // NRB
NCHL = EPW // CHL
E_HALF = E_PAD // 2


def _wid():
    return lax.axis_index("c") * 16 + lax.axis_index("s")


def _sc0_body(h_hbm, src_hbm, dst_hbm, nt_hbm,
              pid_o, hs_o, hd_o, ntg_o,
              hv, sv, dv, pv, hsv, hdv, bufs, bufd, bufn, sema, semb, sem):
    wid = _wid()
    pltpu.sync_copy(h_hbm, hv)
    pltpu.sync_copy(src_hbm.at[wid], sv)
    pltpu.sync_copy(dst_hbm.at[wid], dv)

    def chunk(j, _):
        ga = pltpu.async_copy(h_hbm.at[sv.at[j]], bufs, sema)
        gb = pltpu.async_copy(h_hbm.at[dv.at[j]], bufd, semb)
        ga.wait()
        gb.wait()

        def step(k, _2):
            sl = pl.ds(k * 16, 16)
            hs = bufs[sl]
            hd = bufd[sl]
            o = pl.ds(j * CH + k * 16, 16)
            pv[o] = hs * V + hd
            hsv[o] = hs
            hdv[o] = hd
            return 0

        lax.fori_loop(0, CH // 16, step, 0)
        return 0

    lax.fori_loop(0, NCH, chunk, 0)
    pltpu.sync_copy(pv, pid_o.at[wid])
    pltpu.sync_copy(hsv, hs_o.at[wid])
    pltpu.sync_copy(hdv, hd_o.at[wid])

    nb = wid * NPW
    for off, ln in ((0, 128), (128, 128), (256, 64)):
        pltpu.async_copy(nt_hbm.at[hv.at[pl.ds(nb + off, ln)]],
                         bufn.at[pl.ds(0, ln)], sem).wait()
        pltpu.sync_copy(bufn.at[pl.ds(0, ln)],
                        ntg_o.at[pl.ds(nb + off, ln), :])


def _sc0(h_pad, src3, dst3, nt):
    fn = pl.kernel(
        _sc0_body,
        out_type=(
            jax.ShapeDtypeStruct((NW, EPW), jnp.int32),
            jax.ShapeDtypeStruct((NW, EPW), jnp.int32),
            jax.ShapeDtypeStruct((NW, EPW), jnp.int32),
            jax.ShapeDtypeStruct((N_PAD, 2 * D), F32),
        ),
        mesh=plsc.VectorSubcoreMesh(**_SC_MESH),
        scratch_types=(
            pltpu.VMEM((N_PAD,), jnp.int32),
            pltpu.VMEM((NCH, CH), jnp.int32),
            pltpu.VMEM((NCH, CH), jnp.int32),
            pltpu.VMEM((EPW,), jnp.int32),
            pltpu.VMEM((EPW,), jnp.int32),
            pltpu.VMEM((EPW,), jnp.int32),
            pltpu.VMEM((CH,), jnp.int32),
            pltpu.VMEM((CH,), jnp.int32),
            pltpu.VMEM((128, 2 * D), F32),
            pltpu.SemaphoreType.DMA,
            pltpu.SemaphoreType.DMA,
            pltpu.SemaphoreType.DMA,
        ),
    )
    return fn(h_pad, src3, dst3, nt)


def _sc_l1_body(pid_hbm, dst_hbm, tsg_hbm, z_hbm,
                dn_o,
                pv, dv, buf0, buf1, dn_s, sem0, sem1):
    cid = lax.axis_index("c")
    sid = lax.axis_index("s")
    wid = cid * 16 + sid
    tile_rows = pl.ds(sid * TPS, TPS)
    pltpu.sync_copy(z_hbm, dn_s.at[tile_rows, :])
    plsc.subcore_barrier()

    bufs = (buf0, buf1)
    sems = (sem0, sem1)

    def issue(g, h, s):
        pltpu.async_copy(tsg_hbm.at[pv.at[g, pl.ds(h * 64, 64)]],
                         bufs[s], sems[s])

    def wait_scatter(g, h, s):
        pltpu.make_async_copy(tsg_hbm.at[pv.at[0, pl.ds(0, 64)]],
                              bufs[s], sems[s]).wait()
        pltpu.sync_copy(bufs[s], dn_s.at[dv.at[g, pl.ds(h * 64, 64)]],
                        add=True)

    def blk(b, _):
        pltpu.sync_copy(pid_hbm.at[wid, pl.ds(b * 8, 8)], pv)
        pltpu.sync_copy(dst_hbm.at[wid, pl.ds(b * 8, 8)], dv)
        issue(0, 0, 0)

        def pair(g, _2):
            issue(g, 1, 1)
            wait_scatter(g, 0, 0)

            @pl.when(g < 7)
            def _():
                issue(g + 1, 0, 0)

            wait_scatter(g, 1, 1)
            return 0

        lax.fori_loop(0, 8, pair, 0)
        return 0

    lax.fori_loop(0, NCH // 8, blk, 0)
    plsc.subcore_barrier()
    pltpu.sync_copy(dn_s.at[tile_rows, :], dn_o.at[cid, tile_rows, :])


def _sc_l1(pid3, dst3, tsg, zrow):
    fn = pl.kernel(
        _sc_l1_body,
        out_type=(
            jax.ShapeDtypeStruct((2, N_PAD, 2 * D), F32),
        ),
        mesh=plsc.VectorSubcoreMesh(**_SC_MESH),
        scratch_types=(
            pltpu.VMEM((8, CH), jnp.int32),
            pltpu.VMEM((8, CH), jnp.int32),
            pltpu.VMEM((64, 2 * D), F32),
            pltpu.VMEM((64, 2 * D), F32),
            pltpu.VMEM_SHARED((N_PAD, 2 * D), F32),
            pltpu.SemaphoreType.DMA,
            pltpu.SemaphoreType.DMA,
        ),
    )
    return fn(pid3, dst3, tsg, zrow)[0]


def _make_sc_layer(gather_ce, with_stats):
    def body(*refs):
        it = iter(refs)
        src_hbm = next(it)
        dst_hbm = next(it)
        if gather_ce:
            pid_hbm = next(it)
            tq_hbm = next(it)
        else:
            ce_hbm = next(it)
        dh_hbm = next(it)
        eb_hbm = next(it)
        z_hbm = next(it)
        dn_o = next(it)
        if with_stats:
            ehat_o = next(it)
            ss_o = next(it)
        if gather_ce:
            eein_o = next(it)
        sv = next(it)
        dv = next(it)
        if gather_ce:
            pv = next(it)
            bufi = (next(it), next(it))
        else:
            bufc = (next(it), next(it))
        bufd = (next(it), next(it))
        bufeb = (next(it), next(it))
        bufo = (next(it), next(it))
        bufe = next(it)
        if gather_ce:
            bufw = next(it)
        bufs = next(it)
        dn_s = next(it)
        semd = (next(it), next(it))
        seme = (next(it), next(it))
        semc = (next(it), next(it))

        cid = lax.axis_index("c")
        sid = lax.axis_index("s")
        wid = cid * 16 + sid
        base = wid * EPW
        tile_rows = pl.ds(sid * TPS, TPS)
        pltpu.sync_copy(z_hbm, dn_s.at[tile_rows, :])
        plsc.subcore_barrier()
        nreal = jnp.where(wid == NW - 1, (E - (NW - 1) * EPW) // CHL, NCHL)

        zv = jnp.zeros((16,), F32)

        def issue(b, g, s):
            row = g // 4
            col = (g % 4) * CHL
            pltpu.async_copy(dh_hbm.at[dv.at[row, pl.ds(col, CHL)]],
                             bufd[s], semd[s])
            pltpu.async_copy(eb_hbm.at[sv.at[row, pl.ds(col, CHL)]],
                             bufeb[s], seme[s])
            if gather_ce:
                pltpu.async_copy(tq_hbm.at[pv.at[row, pl.ds(col, CHL)]],
                                 bufi[s], semc[s])
            else:
                off = base + (b * CPB + g) * CHL
                hoff = pl.multiple_of(off // 2, CHL // 2)
                pltpu.async_copy(ce_hbm.at[pl.ds(hoff, CHL // 2), :],
                                 bufc[s], semc[s])

        def wait_set(s):
            pltpu.make_async_copy(
                dh_hbm.at[dv.at[0, pl.ds(0, CHL)]], bufd[s], semd[s]).wait()
            pltpu.make_async_copy(
                eb_hbm.at[sv.at[0, pl.ds(0, CHL)]], bufeb[s], seme[s]).wait()
            if gather_ce:
                pltpu.make_async_copy(
                    tq_hbm.at[pv.at[0, pl.ds(0, CHL)]], bufi[s],
                    semc[s]).wait()
            else:
                pltpu.make_async_copy(
                    ce_hbm.at[pl.ds(0, CHL // 2), :], bufc[s],
                    semc[s]).wait()

        def work(b, g, s, carry):
            row = g // 4
            col = (g % 4) * CHL
            di = dv.at[row, pl.ds(col, CHL)]
            off = base + (b * CPB + g) * CHL
            hoff = pl.multiple_of(off // 2, CHL // 2)
            bd, be_, bo = bufd[s], bufeb[s], bufo[s]
            if gather_ce:
                bi = bufi[s]
            else:
                bc = bufc[s]

            def prow(p, rc):
                out = list(rc)
                for edge in range(2):
                    r = 2 * p + edge
                    for c in range(4):
                        sl = pl.ds(c * 16, 16)
                        sl2 = pl.ds(D + c * 16, 16)
                        pcol = pl.ds(edge * D + c * 16, 16)
                        if gather_ce:
                            ce_v = bi[r, sl]
                        else:
                            ce_v = bc[p, pcol]
                        eh = bd[r, sl] + be_[r, sl] + ce_v
                        bh = be_[r, sl2]
                        sg = 1.0 / (1.0 + jnp.exp(-eh))
                        bo[r, sl] = sg
                        bo[r, sl2] = sg * bh
                        if with_stats:
                            bufe[p, pcol] = eh
                            out[c] = out[c] + eh
                            out[4 + c] = out[4 + c] + eh * eh
                        if gather_ce:
                            bufw[p, pcol] = bi[r, sl2]
                return tuple(out)

            csum = lax.fori_loop(0, CHL // 2, prow, (zv,) * 8)
            pltpu.sync_copy(bo, dn_s.at[di], add=True)
            if with_stats:
                pltpu.sync_copy(bufe, ehat_o.at[pl.ds(hoff, CHL // 2), :])
            if gather_ce:
                pltpu.sync_copy(bufw, eein_o.at[pl.ds(hoff, CHL // 2), :])
            m = jnp.where(b * CPB + g < nreal, 1.0, 0.0).astype(F32)
            return tuple(carry[i] + m * csum[i] for i in range(8))

        def blk(b, bcarry):
            pltpu.sync_copy(src_hbm.at[wid, pl.ds(b * NRB, NRB)], sv)
            pltpu.sync_copy(dst_hbm.at[wid, pl.ds(b * NRB, NRB)], dv)
            if gather_ce:
                pltpu.sync_copy(pid_hbm.at[wid, pl.ds(b * NRB, NRB)], pv)
            issue(b, 0, 0)

            def pairs(gg, carry):
                issue(b, 2 * gg + 1, 1)
                wait_set(0)
                carry = work(b, 2 * gg, 0, carry)

                @pl.when(gg < CPB // 2 - 1)
                def _():
                    issue(b, 2 * gg + 2, 0)

                wait_set(1)
                return work(b, 2 * gg + 1, 1, carry)

            return lax.fori_loop(0, CPB // 2, pairs, bcarry)

        stats = lax.fori_loop(0, NBLKL, blk, (zv,) * 8)
        if with_stats:
            for c in range(4):
                bufs[pl.ds(c * 16, 16)] = stats[c]
                bufs[pl.ds(D + c * 16, 16)] = stats[4 + c]
            pltpu.sync_copy(bufs, ss_o.at[wid])
        plsc.subcore_barrier()
        pltpu.sync_copy(dn_s.at[tile_rows, :], dn_o.at[cid, tile_rows, :])

    out_type = [
        jax.ShapeDtypeStruct((2, N_PAD, 2 * D), F32),
    ]
    if with_stats:
        out_type += [
            jax.ShapeDtypeStruct((E_HALF, 2 * D), F32),
            jax.ShapeDtypeStruct((NW, 2 * D), F32),
        ]
    if gather_ce:
        out_type += [jax.ShapeDtypeStruct((E_HALF, 2 * D), F32)]

    scratch = [
        pltpu.VMEM((NRB, CH), jnp.int32),
        pltpu.VMEM((NRB, CH), jnp.int32),
    ]
    if gather_ce:
        scratch += [pltpu.VMEM((NRB, CH), jnp.int32),
                    pltpu.VMEM((CHL, 2 * D), F32),
                    pltpu.VMEM((CHL, 2 * D), F32)]
    else:
        scratch += [pltpu.VMEM((CHL // 2, 2 * D), F32),
                    pltpu.VMEM((CHL // 2, 2 * D), F32)]
    scratch += [
        pltpu.VMEM((CHL, 2 * D), F32),
        pltpu.VMEM((CHL, 2 * D), F32),
        pltpu.VMEM((CHL, 2 * D), F32),
        pltpu.VMEM((CHL, 2 * D), F32),
        pltpu.VMEM((CHL, 2 * D), F32),
        pltpu.VMEM((CHL, 2 * D), F32),
        pltpu.VMEM((CHL // 2, 2 * D), F32),
    ]
    if gather_ce:
        scratch += [pltpu.VMEM((CHL // 2, 2 * D), F32)]
    scratch += [
        pltpu.VMEM((2 * D,), F32),
        pltpu.VMEM_SHARED((N_PAD, 2 * D), F32),
        pltpu.SemaphoreType.DMA,
        pltpu.SemaphoreType.DMA,
        pltpu.SemaphoreType.DMA,
        pltpu.SemaphoreType.DMA,
        pltpu.SemaphoreType.DMA,
        pltpu.SemaphoreType.DMA,
    ]
    return pl.kernel(
        body, out_type=tuple(out_type),
        mesh=plsc.VectorSubcoreMesh(**_SC_MESH),
        scratch_types=tuple(scratch))


def _dot(a, b):
    return jnp.dot(a, b, preferred_element_type=F32)


def _tab0_body(emb_ref, w1_ref, b1_ref, w2_ref, b2_ref, wetop_ref, webot_ref,
               se_ref, de_ref, a_ref, ab_ref, b_ref, bb_ref, d_ref, db_ref,
               e_ref, eb_ref,
               p1_o, p2_o, l1_o, l2_o, tb_o, td_o, te_o, nt_o):
    emb = emb_ref[...]
    p1_o[...] = _dot(emb, w1_ref[...]) + b1_ref[...]
    p2_o[...] = _dot(emb, w2_ref[...]) + b2_ref[...]
    l1_o[...] = _dot(se_ref[...], wetop_ref[...])
    l2_o[...] = _dot(de_ref[...], webot_ref[...])
    tb_o[...] = _dot(emb, b_ref[...]) + bb_ref[...]
    td_o[...] = _dot(emb, d_ref[...]) + db_ref[...]
    te_o[...] = _dot(emb, e_ref[...]) + eb_ref[...]
    ta = _dot(emb, a_ref[...]) + ab_ref[...]
    nt_o[...] = jnp.concatenate([emb, ta], axis=1)


def _tab0(emb_h, mp, lp0):
    full = pl.BlockSpec(None, lambda: tuple())
    outs = (
        jax.ShapeDtypeStruct((V, D * D), F32),
        jax.ShapeDtypeStruct((V, D), F32),
        jax.ShapeDtypeStruct((V, D), F32),
        jax.ShapeDtypeStruct((V, D), F32),
        jax.ShapeDtypeStruct((V, D), F32),
        jax.ShapeDtypeStruct((V, D), F32),
        jax.ShapeDtypeStruct((V, D), F32),
        jax.ShapeDtypeStruct((V, 2 * D), F32),
    )
    args = (emb_h, mp['W1'], mp['b1'].reshape(1, -1), mp['W2'],
            mp['b2'].reshape(1, -1), mp['W_ep'][:D], mp['W_ep'][D:],
            mp['src_emb'], mp['dst_emb'],
            lp0['A'], lp0['Ab'].reshape(1, -1), lp0['B'],
            lp0['Bb'].reshape(1, -1), lp0['D'], lp0['Db'].reshape(1, -1),
            lp0['E'], lp0['Eb'].reshape(1, -1))
    return pl.pallas_call(
        _tab0_body,
        out_shape=outs,
    )(*args)


_CB = 2000


def _count_body(hs_ref, hd_ref, cnt_o, acc):
    i = pl.program_id(0)

    @pl.when(i == 0)
    def _():
        acc[...] = jnp.zeros_like(acc)

    lanes = lax.broadcasted_iota(jnp.int32, (_CB, V), 1)
    ohs = (hs_ref[...] == lanes).astype(jnp.bfloat16)
    ohd = (hd_ref[...] == lanes).astype(jnp.bfloat16)
    acc[...] += lax.dot_general(ohs, ohd, (((0,), (0,)), ((), ())),
                                preferred_element_type=F32)

    @pl.when(i == pl.num_programs(0) - 1)
    def _():
        cnt_o[...] = acc[...]


def _count(hs_col, hd_col):
    grid = E // _CB
    return pl.pallas_call(
        _count_body,
        grid=(grid,),
        in_specs=[pl.BlockSpec((_CB, 1), lambda i: (i, 0)),
                  pl.BlockSpec((_CB, 1), lambda i: (i, 0))],
        out_specs=pl.BlockSpec((V, V), lambda i: (0, 0)),
        out_shape=jax.ShapeDtypeStruct((V, V), F32),
        scratch_shapes=[pltpu.VMEM((V, V), F32)],
    )(hs_col, hd_col)


def _tab1_body(p1_ref, p2_ref, w3_ref, b3_ref, l1_ref, l2_ref, bep_ref,
               cnt_ref, tx_o, s1_o, s2_o, s1a, s2a):
    u = pl.program_id(0)

    @pl.when(u == 0)
    def _():
        s1a[...] = jnp.zeros_like(s1a)
        s2a[...] = jnp.zeros_like(s2a)

    p1u = p1_ref[0]
    tu = lax.dot_general(p2_ref[...], p1u, (((1,), (1,)), ((), ())),
                         preferred_element_type=F32)
    tx = (_dot(tu, w3_ref[...]) + b3_ref[...] + l1_ref[0]
          + l2_ref[...] + bep_ref[...])
    tx_o[...] = tx
    w = cnt_ref[0] * (1.0 / E)
    s1a[...] += _dot(w, tx)
    s2a[...] += _dot(w, tx * tx)

    @pl.when(u == pl.num_programs(0) - 1)
    def _():
        s1_o[...] = s1a[...]
        s2_o[...] = s2a[...]


def _tab1(p1v3, p2v, mp, l1t, l2t, cnt):
    return pl.pallas_call(
        _tab1_body,
        grid=(V,),
        in_specs=[
            pl.BlockSpec((1, D, D), lambda u: (u, 0, 0)),
            pl.BlockSpec((V, D), lambda u: (0, 0)),
            pl.BlockSpec((D, D), lambda u: (0, 0)),
            pl.BlockSpec((1, D), lambda u: (0, 0)),
            pl.BlockSpec((1, 1, D), lambda u: (u, 0, 0)),
            pl.BlockSpec((V, D), lambda u: (0, 0)),
            pl.BlockSpec((1, D), lambda u: (0, 0)),
            pl.BlockSpec((1, 1, V), lambda u: (u, 0, 0)),
        ],
        out_specs=[
            pl.BlockSpec((V, D), lambda u: (u, 0)),
            pl.BlockSpec((1, D), lambda u: (0, 0)),
            pl.BlockSpec((1, D), lambda u: (0, 0)),
        ],
        out_shape=[
            jax.ShapeDtypeStruct((V * V, D), F32),
            jax.ShapeDtypeStruct((1, D), F32),
            jax.ShapeDtypeStruct((1, D), F32),
        ],
        scratch_shapes=[pltpu.VMEM((1, D), F32), pltpu.VMEM((1, D), F32)],
    )(p1v3, p2v, mp['W_ep3'], mp['b_ep3'].reshape(1, -1),
      l1t.reshape(V, 1, D), l2t, mp['b_ep'].reshape(1, -1),
      cnt.reshape(V, 1, V))


def _tab2_body(tx_ref, s1_ref, s2_ref, g_ref, b_ref, td_ref, te_ref, tb_ref,
               c_ref, cb_ref, cnt_ref,
               tee_o, teh_o, tsg_o, s1e_o, s2e_o, s1a, s2a):
    u = pl.program_id(0)

    @pl.when(u == 0)
    def _():
        s1a[...] = jnp.zeros_like(s1a)
        s2a[...] = jnp.zeros_like(s2a)

    mu = s1_ref[...]
    inv = lax.rsqrt(s2_ref[...] - mu * mu + EPS)
    tee = jnp.maximum((tx_ref[...] - mu) * inv * g_ref[...] + b_ref[...], 0.0)
    tee_o[...] = tee
    teh = _dot(tee, c_ref[...]) + cb_ref[...] + td_ref[...] + te_ref[0]
    teh_o[...] = teh
    tsig = jax.nn.sigmoid(teh)
    tsg_o[...] = jnp.concatenate([tsig, tsig * tb_ref[0]], axis=1)
    w = cnt_ref[0] * (1.0 / E)
    s1a[...] += _dot(w, teh)
    s2a[...] += _dot(w, teh * teh)

    @pl.when(u == pl.num_programs(0) - 1)
    def _():
        s1e_o[...] = s1a[...]
        s2e_o[...] = s2a[...]


def _tab2(tx, s1m, s2m, mp, lp0, tbt, tdt, tet, cnt):
    return pl.pallas_call(
        _tab2_body,
        grid=(V,),
        in_specs=[
            pl.BlockSpec((V, D), lambda u: (u, 0)),
            pl.BlockSpec((1, D), lambda u: (0, 0)),
            pl.BlockSpec((1, D), lambda u: (0, 0)),
            pl.BlockSpec((1, D), lambda u: (0, 0)),
            pl.BlockSpec((1, D), lambda u: (0, 0)),
            pl.BlockSpec((V, D), lambda u: (0, 0)),
            pl.BlockSpec((1, 1, D), lambda u: (u, 0, 0)),
            pl.BlockSpec((1, 1, D), lambda u: (u, 0, 0)),
            pl.BlockSpec((D, D), lambda u: (0, 0)),
            pl.BlockSpec((1, D), lambda u: (0, 0)),
            pl.BlockSpec((1, 1, V), lambda u: (u, 0, 0)),
        ],
        out_specs=[
            pl.BlockSpec((V, D), lambda u: (u, 0)),
            pl.BlockSpec((V, D), lambda u: (u, 0)),
            pl.BlockSpec((V, 2 * D), lambda u: (u, 0)),
            pl.BlockSpec((1, D), lambda u: (0, 0)),
            pl.BlockSpec((1, D), lambda u: (0, 0)),
        ],
        out_shape=[
            jax.ShapeDtypeStruct((V * V, D), F32),
            jax.ShapeDtypeStruct((V * V, D), F32),
            jax.ShapeDtypeStruct((V * V, 2 * D), F32),
            jax.ShapeDtypeStruct((1, D), F32),
            jax.ShapeDtypeStruct((1, D), F32),
        ],
        scratch_shapes=[pltpu.VMEM((1, D), F32), pltpu.VMEM((1, D), F32)],
    )(tx, s1m, s2m, mp['bn_g'].reshape(1, -1), mp['bn_b'].reshape(1, -1),
      tdt, tet.reshape(V, 1, D), tbt.reshape(V, 1, D), lp0['C'],
      lp0['Cb'].reshape(1, -1), cnt.reshape(V, 1, V))


def _tab3_body(tee_ref, teh_ref, s1_ref, s2_ref, g_ref, b_ref, c_ref, cb_ref,
               tq_o):
    mu = s1_ref[...]
    inv = lax.rsqrt(s2_ref[...] - mu * mu + EPS)
    te2 = tee_ref[...] + jnp.maximum(
        (teh_ref[...] - mu) * inv * g_ref[...] + b_ref[...], 0.0)
    tq_o[...] = jnp.concatenate(
        [_dot(te2, c_ref[...]) + cb_ref[...], te2], axis=1)


def _tab3(tee, teh, s1e, s2e, lp0, lp1):
    return pl.pallas_call(
        _tab3_body,
        grid=(V,),
        in_specs=[
            pl.BlockSpec((V, D), lambda u: (u, 0)),
            pl.BlockSpec((V, D), lambda u: (u, 0)),
            pl.BlockSpec((1, D), lambda u: (0, 0)),
            pl.BlockSpec((1, D), lambda u: (0, 0)),
            pl.BlockSpec((1, D), lambda u: (0, 0)),
            pl.BlockSpec((1, D), lambda u: (0, 0)),
            pl.BlockSpec((D, D), lambda u: (0, 0)),
            pl.BlockSpec((1, D), lambda u: (0, 0)),
        ],
        out_specs=pl.BlockSpec((V, 2 * D), lambda u: (u, 0)),
        out_shape=jax.ShapeDtypeStruct((V * V, 2 * D), F32),
    )(tee, teh, s1e, s2e, lp0['bne_g'].reshape(1, -1),
      lp0['bne_b'].reshape(1, -1), lp1['C'], lp1['Cb'].reshape(1, -1))


_NB = 2048


def _row_mask(nrows, limit):
    rows = (pl.program_id(0) * nrows
            + lax.broadcasted_iota(jnp.int32, (nrows, 1), 0))
    return rows < limit


def _mm_body(hh_ref, a_ref, ab_ref, b_ref, bb_ref, d_ref, db_ref,
             e_ref, eb_ref, ah_o, dh_o, ebo_o):
    m = _row_mask(_NB, N)
    hh = hh_ref[...]
    ah_o[...] = jnp.where(m, _dot(hh, a_ref[...]) + ab_ref[...], 0.0)
    dh = jnp.where(m, _dot(hh, d_ref[...]) + db_ref[...], 0.0)
    dh_o[...] = jnp.concatenate([dh, jnp.zeros_like(dh)], axis=1)
    eh = _dot(hh, e_ref[...]) + eb_ref[...]
    bh = _dot(hh, b_ref[...]) + bb_ref[...]
    ebo_o[...] = jnp.where(m, jnp.concatenate([eh, bh], axis=1), 0.0)


def _tc_mm(hh, lp):
    return pl.pallas_call(
        _mm_body,
        grid=(N_PAD // _NB,),
        in_specs=[pl.BlockSpec((_NB, D), lambda i: (i, 0))]
        + [pl.BlockSpec((D, D), lambda i: (0, 0)),
           pl.BlockSpec((1, D), lambda i: (0, 0))] * 4,
        out_specs=[
            pl.BlockSpec((_NB, D), lambda i: (i, 0)),
            pl.BlockSpec((_NB, 2 * D), lambda i: (i, 0)),
            pl.BlockSpec((_NB, 2 * D), lambda i: (i, 0)),
        ],
        out_shape=[
            jax.ShapeDtypeStruct((N_PAD, D), F32),
            jax.ShapeDtypeStruct((N_PAD, 2 * D), F32),
            jax.ShapeDtypeStruct((N_PAD, 2 * D), F32),
        ],
    )(hh, lp['A'], lp['Ab'].reshape(1, -1), lp['B'], lp['Bb'].reshape(1, -1),
      lp['D'], lp['Db'].reshape(1, -1), lp['E'], lp['Eb'].reshape(1, -1))


def _ha_body(ah_ref, dn_ref, hn_o, s1_o, s2_o, s1a, s2a):
    i = pl.program_id(0)

    @pl.when(i == 0)
    def _():
        s1a[...] = jnp.zeros_like(s1a)
        s2a[...] = jnp.zeros_like(s2a)

    dn = dn_ref[0] + dn_ref[1]
    den = dn[:, :D]
    num = dn[:, D:]
    hn = ah_ref[...] + num / (den + 1e-6)
    hn = jnp.where(_row_mask(_NB, N), hn, 0.0)
    hn_o[...] = hn
    s1a[...] += jnp.sum(hn, axis=0, keepdims=True)
    s2a[...] += jnp.sum(hn * hn, axis=0, keepdims=True)

    @pl.when(i == pl.num_programs(0) - 1)
    def _():
        s1_o[...] = s1a[...]
        s2_o[...] = s2a[...]


def _tc_ha(ah, dn_p):
    return pl.pallas_call(
        _ha_body,
        grid=(N_PAD // _NB,),
        in_specs=[
            pl.BlockSpec((_NB, D), lambda i: (i, 0)),
            pl.BlockSpec((2, _NB, 2 * D), lambda i: (0, i, 0)),
        ],
        out_specs=[
            pl.BlockSpec((_NB, D), lambda i: (i, 0)),
            pl.BlockSpec((1, D), lambda i: (0, 0)),
            pl.BlockSpec((1, D), lambda i: (0, 0)),
        ],
        out_shape=[
            jax.ShapeDtypeStruct((N_PAD, D), F32),
            jax.ShapeDtypeStruct((1, D), F32),
            jax.ShapeDtypeStruct((1, D), F32),
        ],
        scratch_shapes=[pltpu.VMEM((1, D), F32), pltpu.VMEM((1, D), F32)],
    )(ah, dn_p)


def _hb_body(hh_ref, hn_ref, s1_ref, s2_ref, g_ref, b_ref, hho_o):
    mu = s1_ref[...] * (1.0 / N)
    var = s2_ref[...] * (1.0 / N) - mu * mu
    inv = lax.rsqrt(var + EPS)
    hho = hh_ref[...] + jnp.maximum(
        (hn_ref[...] - mu) * inv * g_ref[...] + b_ref[...], 0.0)
    hho_o[...] = jnp.where(_row_mask(_NB, N), hho, 0.0)


def _tc_hb(hh, hn, s1, s2, lp):
    return pl.pallas_call(
        _hb_body,
        grid=(N_PAD // _NB,),
        in_specs=[
            pl.BlockSpec((_NB, D), lambda i: (i, 0)),
            pl.BlockSpec((_NB, D), lambda i: (i, 0)),
            pl.BlockSpec((1, D), lambda i: (0, 0)),
            pl.BlockSpec((1, D), lambda i: (0, 0)),
            pl.BlockSpec((1, D), lambda i: (0, 0)),
            pl.BlockSpec((1, D), lambda i: (0, 0)),
        ],
        out_specs=pl.BlockSpec((_NB, D), lambda i: (i, 0)),
        out_shape=jax.ShapeDtypeStruct((N_PAD, D), F32),
    )(hh, hn, s1, s2, lp['bnh_g'].reshape(1, -1), lp['bnh_b'].reshape(1, -1))


_EB = 2048


def _make_e_body(with_ce):
    def body(*refs):
        if with_ce:
            (eh_ref, ee_ref, ss_ref, g_ref, b_ref, c_ref, cb_ref,
             eo_o, ce_o) = refs
        else:
            (eh_ref, ee_ref, ss_ref, g_ref, b_ref, eo_o) = refs
        ss = jnp.sum(ss_ref[...], axis=0, keepdims=True) * (1.0 / E)
        mu = ss[:, :D]
        inv = lax.rsqrt(ss[:, D:] - mu * mu + EPS)
        mu2 = jnp.concatenate([mu, mu], axis=1)
        inv2 = jnp.concatenate([inv, inv], axis=1)
        g2 = jnp.concatenate([g_ref[...], g_ref[...]], axis=1)
        b2 = jnp.concatenate([b_ref[...], b_ref[...]], axis=1)
        eo = ee_ref[...] + jnp.maximum(
            (eh_ref[...] - mu2) * inv2 * g2 + b2, 0.0)
        m = _row_mask(_EB, E // 2)
        eo = jnp.where(m, eo, 0.0)
        eo_o[...] = eo
        if with_ce:
            ce_o[...] = _dot(eo, c_ref[...]) + cb_ref[...]
    return body


def _tc_e(ehat, eein, ss, lp, lp_next):
    with_ce = lp_next is not None
    in_specs = [
        pl.BlockSpec((_EB, 2 * D), lambda i: (i, 0)),
        pl.BlockSpec((_EB, 2 * D), lambda i: (i, 0)),
        pl.BlockSpec((NW, 2 * D), lambda i: (0, 0)),
        pl.BlockSpec((1, D), lambda i: (0, 0)),
        pl.BlockSpec((1, D), lambda i: (0, 0)),
    ]
    args = [ehat, eein, ss, lp['bne_g'].reshape(1, -1),
            lp['bne_b'].reshape(1, -1)]
    out_specs = [pl.BlockSpec((_EB, 2 * D), lambda i: (i, 0))]
    out_shape = [jax.ShapeDtypeStruct((E_HALF, 2 * D), F32)]
    if with_ce:
        zdd = jnp.zeros((D, D), F32)
        c2 = jnp.block([[lp_next['C'], zdd], [zdd, lp_next['C']]])
        cb2 = jnp.tile(lp_next['Cb'], 2).reshape(1, -1)
        in_specs += [pl.BlockSpec((2 * D, 2 * D), lambda i: (0, 0)),
                     pl.BlockSpec((1, 2 * D), lambda i: (0, 0))]
        args += [c2, cb2]
        out_specs += [pl.BlockSpec((_EB, 2 * D), lambda i: (i, 0))]
        out_shape += [jax.ShapeDtypeStruct((E_HALF, 2 * D), F32)]
    res = pl.pallas_call(
        _make_e_body(with_ce),
        grid=(E_HALF // _EB,),
        in_specs=in_specs,
        out_specs=out_specs,
        out_shape=out_shape,
    )(*args)
    return res if with_ce else (res[0], None)


def _mlp_body(hh_ref, w0_ref, b0_ref, w1_ref, b1_ref, w2_ref, b2_ref, y_o):
    y = jnp.maximum(_dot(hh_ref[...], w0_ref[...]) + b0_ref[...], 0.0)
    y = jnp.maximum(_dot(y, w1_ref[...]) + b1_ref[...], 0.0)
    y_o[...] = _dot(y, w2_ref[...]) + b2_ref[...]


def _tc_mlp(hh, mlp):
    d0 = mlp['W0'].shape[1]
    d1 = mlp['W1'].shape[1]
    d2 = mlp['W2'].shape[1]
    return pl.pallas_call(
        _mlp_body,
        grid=(N_PAD // _NB,),
        in_specs=[
            pl.BlockSpec((_NB, D), lambda i: (i, 0)),
            pl.BlockSpec((D, d0), lambda i: (0, 0)),
            pl.BlockSpec((1, d0), lambda i: (0, 0)),
            pl.BlockSpec((d0, d1), lambda i: (0, 0)),
            pl.BlockSpec((1, d1), lambda i: (0, 0)),
            pl.BlockSpec((d1, d2), lambda i: (0, 0)),
            pl.BlockSpec((1, d2), lambda i: (0, 0)),
        ],
        out_specs=pl.BlockSpec((_NB, d2), lambda i: (i, 0)),
        out_shape=jax.ShapeDtypeStruct((N_PAD, d2), F32),
    )(hh, mlp['W0'], mlp['b0'].reshape(1, -1), mlp['W1'],
      mlp['b1'].reshape(1, -1), mlp['W2'], mlp['b2'].reshape(1, -1))


def kernel(h, edge_index, e, emb_h, gtp, merg, layers, mlp):
    del e, gtp
    h = h.astype(jnp.int32)
    src = edge_index[0].astype(jnp.int32)
    dst = edge_index[1].astype(jnp.int32)

    h_pad = jnp.concatenate([h, jnp.zeros((N_PAD - N,), jnp.int32)])
    pad_e = jnp.full((E_PAD - E,), DUMMY, jnp.int32)
    src2 = jnp.concatenate([src, pad_e]).reshape(NW, EPW)
    dst2 = jnp.concatenate([dst, pad_e]).reshape(NW, EPW)
    zrow = jnp.zeros((TPS, 2 * D), F32)

    lp = layers

    p1v, p2v, l1t, l2t, tbt, tdt, tet, nt = _tab0(emb_h, merg, lp[0])
    src3 = src2.reshape(NW, NCH, CH)
    dst3 = dst2.reshape(NW, NCH, CH)
    pid2, hs2, hd2, ntg = _sc0(h_pad, src3, dst3, nt)
    pid3 = pid2.reshape(NW, NCH, CH)
    hs_col = hs2.reshape(E_PAD, 1)[:E]
    hd_col = hd2.reshape(E_PAD, 1)[:E]

    cnt = _count(hs_col, hd_col)
    tx, s1m, s2m = _tab1(p1v.reshape(V, D, D), p2v, merg, l1t, l2t, cnt)
    tee, teh, tsg, s1e1, s2e1 = _tab2(
        tx, s1m, s2m, merg, lp[0], tbt, tdt, tet, cnt)
    tq = _tab3(tee, teh, s1e1, s2e1, lp[0], lp[1])

    dn_p = _sc_l1(pid3, dst3, tsg, zrow)
    hemb = ntg[:, :D]
    ah1 = ntg[:, D:]
    hn, s1h, s2h = _tc_ha(ah1, dn_p)
    hh = _tc_hb(hemb, hn, s1h, s2h, lp[0])

    sc_l2 = _make_sc_layer(gather_ce=True, with_stats=True)
    sc_l3 = _make_sc_layer(gather_ce=False, with_stats=True)
    sc_l4 = _make_sc_layer(gather_ce=False, with_stats=False)

    ah, dh, ebt = _tc_mm(hh, lp[1])
    dn_p, ehat, ss, eein = sc_l2(src3, dst3, pid3, tq, dh, ebt, zrow)
    hn, s1h, s2h = _tc_ha(ah, dn_p)
    hh = _tc_hb(hh, hn, s1h, s2h, lp[1])
    ee, ce = _tc_e(ehat, eein, ss, lp[1], lp[2])

    ah, dh, ebt = _tc_mm(hh, lp[2])
    dn_p, ehat, ss = sc_l3(src3, dst3, ce, dh, ebt, zrow)
    hn, s1h, s2h = _tc_ha(ah, dn_p)
    hh = _tc_hb(hh, hn, s1h, s2h, lp[2])
    ee, ce = _tc_e(ehat, ee, ss, lp[2], lp[3])

    ah, dh, ebt = _tc_mm(hh, lp[3])
    dn_p, = sc_l4(src3, dst3, ce, dh, ebt, zrow)
    hn, s1h, s2h = _tc_ha(ah, dn_p)
    hh = _tc_hb(hh, hn, s1h, s2h, lp[3])

    return _tc_mlp(hh, mlp)[:N]


# speedup vs baseline: 5.6381x; 1.0295x over previous
import functools

import jax
import jax.numpy as jnp
from jax import lax
from jax.experimental import pallas as pl
from jax.experimental.pallas import tpu as pltpu
from jax.experimental.pallas import tpu_sc as plsc

F32 = jnp.float32
EPS = 1e-5

N = 10000
E = 320000
D = 64
V = 128
NW = 32
N_PAD = 10240
E_PAD = 327680
EPW = E_PAD // NW
NCH = 80
CH = 128
NPW = N_PAD // NW
TPS = N_PAD // 16
DUMMY = N

_SC_MESH = dict(core_axis_name="c", subcore_axis_name="s")

CHL = 32
NRB = 4
CPB = NRB * (CH // CHL)
NBLKL = NCH // NRB
NCHL = EPW // CHL
E_HALF = E_PAD // 2


def _wid():
    return lax.axis_index("c") * 16 + lax.axis_index("s")


def _sc0_body(h_hbm, src_hbm, dst_hbm, nt_hbm,
              pid_o, hs_o, hd_o, ntg_o,
              hv, sv, dv, pv, hsv, hdv, bufs, bufd, bufn, sema, semb, sem):
    wid = _wid()
    pltpu.sync_copy(h_hbm, hv)
    pltpu.sync_copy(src_hbm.at[wid], sv)
    pltpu.sync_copy(dst_hbm.at[wid], dv)

    def chunk(j, _):
        ga = pltpu.async_copy(h_hbm.at[sv.at[j]], bufs, sema)
        gb = pltpu.async_copy(h_hbm.at[dv.at[j]], bufd, semb)
        ga.wait()
        gb.wait()

        def step(k, _2):
            sl = pl.ds(k * 16, 16)
            hs = bufs[sl]
            hd = bufd[sl]
            o = pl.ds(j * CH + k * 16, 16)
            pv[o] = hs * V + hd
            hsv[o] = hs
            hdv[o] = hd
            return 0

        lax.fori_loop(0, CH // 16, step, 0)
        return 0

    lax.fori_loop(0, NCH, chunk, 0)
    pltpu.sync_copy(pv, pid_o.at[wid])
    pltpu.sync_copy(hsv, hs_o.at[wid])
    pltpu.sync_copy(hdv, hd_o.at[wid])

    nb = wid * NPW
    for off, ln in ((0, 128), (128, 128), (256, 64)):
        pltpu.async_copy(nt_hbm.at[hv.at[pl.ds(nb + off, ln)]],
                         bufn.at[pl.ds(0, ln)], sem).wait()
        pltpu.sync_copy(bufn.at[pl.ds(0, ln)],
                        ntg_o.at[pl.ds(nb + off, ln), :])


def _sc0(h_pad, src3, dst3, nt):
    fn = pl.kernel(
        _sc0_body,
        out_type=(
            jax.ShapeDtypeStruct((NW, EPW), jnp.int32),
            jax.ShapeDtypeStruct((NW, EPW), jnp.int32),
            jax.ShapeDtypeStruct((NW, EPW), jnp.int32),
            jax.ShapeDtypeStruct((N_PAD, 2 * D), F32),
        ),
        mesh=plsc.VectorSubcoreMesh(**_SC_MESH),
        scratch_types=(
            pltpu.VMEM((N_PAD,), jnp.int32),
            pltpu.VMEM((NCH, CH), jnp.int32),
            pltpu.VMEM((NCH, CH), jnp.int32),
            pltpu.VMEM((EPW,), jnp.int32),
            pltpu.VMEM((EPW,), jnp.int32),
            pltpu.VMEM((EPW,), jnp.int32),
            pltpu.VMEM((CH,), jnp.int32),
            pltpu.VMEM((CH,), jnp.int32),
            pltpu.VMEM((128, 2 * D), F32),
            pltpu.SemaphoreType.DMA,
            pltpu.SemaphoreType.DMA,
            pltpu.SemaphoreType.DMA,
        ),
    )
    return fn(h_pad, src3, dst3, nt)


def _sc_l1_body(pid_hbm, dst_hbm, tsg_hbm, z_hbm,
                dn_o,
                pv, dv, buf0, buf1, dn_s, semg0, semg1, semo0, semo1):
    cid = lax.axis_index("c")
    sid = lax.axis_index("s")
    wid = cid * 16 + sid
    tile_rows = pl.ds(sid * TPS, TPS)
    pltpu.sync_copy(z_hbm, dn_s.at[tile_rows, :])
    plsc.subcore_barrier()

    bufs = (buf0, buf1)
    sems = (semg0, semg1)
    semo = (semo0, semo1)

    def issue(k, g, h, s):
        @pl.when(k >= 2)
        def _():
            pltpu.make_async_copy(
                bufs[s], dn_s.at[dv.at[0, pl.ds(0, 64)]], semo[s]).wait()

        pltpu.async_copy(tsg_hbm.at[pv.at[g, pl.ds(h * 64, 64)]],
                         bufs[s], sems[s])

    def proc(g, h, s):
        pltpu.make_async_copy(tsg_hbm.at[pv.at[0, pl.ds(0, 64)]],
                              bufs[s], sems[s]).wait()
        pltpu.async_copy(bufs[s], dn_s.at[dv.at[g, pl.ds(h * 64, 64)]],
                         semo[s], add=True)

    def blk(b, _):
        pltpu.sync_copy(pid_hbm.at[wid, pl.ds(b * 8, 8)], pv)
        pltpu.sync_copy(dst_hbm.at[wid, pl.ds(b * 8, 8)], dv)
        issue(b * 16, 0, 0, 0)

        def pair(g, _2):
            issue(b * 16 + 2 * g + 1, g, 1, 1)
            proc(g, 0, 0)

            @pl.when(g < 7)
            def _():
                issue(b * 16 + 2 * g + 2, g + 1, 0, 0)

            proc(g, 1, 1)
            return 0

        lax.fori_loop(0, 8, pair, 0)
        return 0

    lax.fori_loop(0, NCH // 8, blk, 0)
    for s in (0, 1):
        pltpu.make_async_copy(
            bufs[s], dn_s.at[dv.at[0, pl.ds(0, 64)]], semo[s]).wait()
    plsc.subcore_barrier()
    pltpu.sync_copy(dn_s.at[tile_rows, :], dn_o.at[cid, tile_rows, :])


def _sc_l1(pid3, dst3, tsg, zrow):
    fn = pl.kernel(
        _sc_l1_body,
        out_type=(
            jax.ShapeDtypeStruct((2, N_PAD, 2 * D), F32),
        ),
        mesh=plsc.VectorSubcoreMesh(**_SC_MESH),
        scratch_types=(
            pltpu.VMEM((8, CH), jnp.int32),
            pltpu.VMEM((8, CH), jnp.int32),
            pltpu.VMEM((64, 2 * D), F32),
            pltpu.VMEM((64, 2 * D), F32),
            pltpu.VMEM_SHARED((N_PAD, 2 * D), F32),
            pltpu.SemaphoreType.DMA,
            pltpu.SemaphoreType.DMA,
            pltpu.SemaphoreType.DMA,
            pltpu.SemaphoreType.DMA,
        ),
    )
    return fn(pid3, dst3, tsg, zrow)[0]


def _make_sc_layer(gather_ce, with_stats):
    def body(*refs):
        it = iter(refs)
        src_hbm = next(it)
        dst_hbm = next(it)
        if gather_ce:
            pid_hbm = next(it)
            tq_hbm = next(it)
        else:
            ce_hbm = next(it)
        dh_hbm = next(it)
        eb_hbm = next(it)
        z_hbm = next(it)
        dn_o = next(it)
        if with_stats:
            ehat_o = next(it)
            ss_o = next(it)
        if gather_ce:
            eein_o = next(it)
        sv = next(it)
        dv = next(it)
        if gather_ce:
            pv = next(it)
            bufi = (next(it), next(it))
        else:
            bufc = (next(it), next(it))
        bufd = (next(it), next(it))
        bufeb = (next(it), next(it))
        bufo = (next(it), next(it))
        bufe = next(it)
        if gather_ce:
            bufw = next(it)
        bufs = next(it)
        dn_s = next(it)
        semd = (next(it), next(it))
        seme = (next(it), next(it))
        semc = (next(it), next(it))
        semo = (next(it), next(it))

        cid = lax.axis_index("c")
        sid = lax.axis_index("s")
        wid = cid * 16 + sid
        base = wid * EPW
        tile_rows = pl.ds(sid * TPS, TPS)
        pltpu.sync_copy(z_hbm, dn_s.at[tile_rows, :])
        plsc.subcore_barrier()
        nreal = jnp.where(wid == NW - 1, (E - (NW - 1) * EPW) // CHL, NCHL)

        zv = jnp.zeros((16,), F32)

        def issue(b, g, s):
            row = g // 4
            col = (g % 4) * CHL
            pltpu.async_copy(dh_hbm.at[dv.at[row, pl.ds(col, CHL)]],
                             bufd[s], semd[s])
            pltpu.async_copy(eb_hbm.at[sv.at[row, pl.ds(col, CHL)]],
                             bufeb[s], seme[s])
            if gather_ce:
                pltpu.async_copy(tq_hbm.at[pv.at[row, pl.ds(col, CHL)]],
                                 bufi[s], semc[s])
            else:
                off = base + (b * CPB + g) * CHL
                hoff = pl.multiple_of(off // 2, CHL // 2)
                pltpu.async_copy(ce_hbm.at[pl.ds(hoff, CHL // 2), :],
                                 bufc[s], semc[s])

        def wait_set(s):
            pltpu.make_async_copy(
                dh_hbm.at[dv.at[0, pl.ds(0, CHL)]], bufd[s], semd[s]).wait()
            pltpu.make_async_copy(
                eb_hbm.at[sv.at[0, pl.ds(0, CHL)]], bufeb[s], seme[s]).wait()
            if gather_ce:
                pltpu.make_async_copy(
                    tq_hbm.at[pv.at[0, pl.ds(0, CHL)]], bufi[s],
                    semc[s]).wait()
            else:
                pltpu.make_async_copy(
                    ce_hbm.at[pl.ds(0, CHL // 2), :], bufc[s],
                    semc[s]).wait()

        def work(b, g, s, carry):
            row = g // 4
            col = (g % 4) * CHL
            di = dv.at[row, pl.ds(col, CHL)]
            off = base + (b * CPB + g) * CHL
            hoff = pl.multiple_of(off // 2, CHL // 2)
            bd, be_, bo = bufd[s], bufeb[s], bufo[s]

            @pl.when(b * CPB + g >= 2)
            def _():
                pltpu.make_async_copy(
                    bo, dn_s.at[dv.at[0, pl.ds(0, CHL)]], semo[s]).wait()
            if gather_ce:
                bi = bufi[s]
            else:
                bc = bufc[s]

            def prow(p, rc):
                out = list(rc)
                for edge in range(2):
                    r = 2 * p + edge
                    for c in range(4):
                        sl = pl.ds(c * 16, 16)
                        sl2 = pl.ds(D + c * 16, 16)
                        pcol = pl.ds(edge * D + c * 16, 16)
                        if gather_ce:
                            ce_v = bi[r, sl]
                        else:
                            ce_v = bc[p, pcol]
                        eh = bd[r, sl] + be_[r, sl] + ce_v
                        bh = be_[r, sl2]
                        sg = 1.0 / (1.0 + jnp.exp(-eh))
                        bo[r, sl] = sg
                        bo[r, sl2] = sg * bh
                        if with_stats:
                            bufe[p, pcol] = eh
                            out[c] = out[c] + eh
                            out[4 + c] = out[4 + c] + eh * eh
                        if gather_ce:
                            bufw[p, pcol] = bi[r, sl2]
                return tuple(out)

            csum = lax.fori_loop(0, CHL // 2, prow, (zv,) * 8)
            pltpu.async_copy(bo, dn_s.at[di], semo[s], add=True)
            if with_stats:
                pltpu.sync_copy(bufe, ehat_o.at[pl.ds(hoff, CHL // 2), :])
            if gather_ce:
                pltpu.sync_copy(bufw, eein_o.at[pl.ds(hoff, CHL // 2), :])
            m = jnp.where(b * CPB + g < nreal, 1.0, 0.0).astype(F32)
            return tuple(carry[i] + m * csum[i] for i in range(8))

        def blk(b, bcarry):
            pltpu.sync_copy(src_hbm.at[wid, pl.ds(b * NRB, NRB)], sv)
            pltpu.sync_copy(dst_hbm.at[wid, pl.ds(b * NRB, NRB)], dv)
            if gather_ce:
                pltpu.sync_copy(pid_hbm.at[wid, pl.ds(b * NRB, NRB)], pv)
            issue(b, 0, 0)

            def pairs(gg, carry):
                issue(b, 2 * gg + 1, 1)
                wait_set(0)
                carry = work(b, 2 * gg, 0, carry)

                @pl.when(gg < CPB // 2 - 1)
                def _():
                    issue(b, 2 * gg + 2, 0)

                wait_set(1)
                return work(b, 2 * gg + 1, 1, carry)

            return lax.fori_loop(0, CPB // 2, pairs, bcarry)

        stats = lax.fori_loop(0, NBLKL, blk, (zv,) * 8)
        for s in (0, 1):
            pltpu.make_async_copy(
                bufo[s], dn_s.at[dv.at[0, pl.ds(0, CHL)]], semo[s]).wait()
        if with_stats:
            for c in range(4):
                bufs[pl.ds(c * 16, 16)] = stats[c]
                bufs[pl.ds(D + c * 16, 16)] = stats[4 + c]
            pltpu.sync_copy(bufs, ss_o.at[wid])
        plsc.subcore_barrier()
        pltpu.sync_copy(dn_s.at[tile_rows, :], dn_o.at[cid, tile_rows, :])

    out_type = [
        jax.ShapeDtypeStruct((2, N_PAD, 2 * D), F32),
    ]
    if with_stats:
        out_type += [
            jax.ShapeDtypeStruct((E_HALF, 2 * D), F32),
            jax.ShapeDtypeStruct((NW, 2 * D), F32),
        ]
    if gather_ce:
        out_type += [jax.ShapeDtypeStruct((E_HALF, 2 * D), F32)]

    scratch = [
        pltpu.VMEM((NRB, CH), jnp.int32),
        pltpu.VMEM((NRB, CH), jnp.int32),
    ]
    if gather_ce:
        scratch += [pltpu.VMEM((NRB, CH), jnp.int32),
                    pltpu.VMEM((CHL, 2 * D), F32),
                    pltpu.VMEM((CHL, 2 * D), F32)]
    else:
        scratch += [pltpu.VMEM((CHL // 2, 2 * D), F32),
                    pltpu.VMEM((CHL // 2, 2 * D), F32)]
    scratch += [
        pltpu.VMEM((CHL, 2 * D), F32),
        pltpu.VMEM((CHL, 2 * D), F32),
        pltpu.VMEM((CHL, 2 * D), F32),
        pltpu.VMEM((CHL, 2 * D), F32),
        pltpu.VMEM((CHL, 2 * D), F32),
        pltpu.VMEM((CHL, 2 * D), F32),
        pltpu.VMEM((CHL // 2, 2 * D), F32),
    ]
    if gather_ce:
        scratch += [pltpu.VMEM((CHL // 2, 2 * D), F32)]
    scratch += [
        pltpu.VMEM((2 * D,), F32),
        pltpu.VMEM_SHARED((N_PAD, 2 * D), F32),
        pltpu.SemaphoreType.DMA,
        pltpu.SemaphoreType.DMA,
        pltpu.SemaphoreType.DMA,
        pltpu.SemaphoreType.DMA,
        pltpu.SemaphoreType.DMA,
        pltpu.SemaphoreType.DMA,
        pltpu.SemaphoreType.DMA,
        pltpu.SemaphoreType.DMA,
    ]
    return pl.kernel(
        body, out_type=tuple(out_type),
        mesh=plsc.VectorSubcoreMesh(**_SC_MESH),
        scratch_types=tuple(scratch))


def _dot(a, b):
    return jnp.dot(a, b, preferred_element_type=F32)


def _tab0_body(emb_ref, w1_ref, b1_ref, w2_ref, b2_ref, wetop_ref, webot_ref,
               se_ref, de_ref, a_ref, ab_ref, b_ref, bb_ref, d_ref, db_ref,
               e_ref, eb_ref,
               p1_o, p2_o, l1_o, l2_o, tb_o, td_o, te_o, nt_o):
    emb = emb_ref[...]
    p1_o[...] = _dot(emb, w1_ref[...]) + b1_ref[...]
    p2_o[...] = _dot(emb, w2_ref[...]) + b2_ref[...]
    l1_o[...] = _dot(se_ref[...], wetop_ref[...])
    l2_o[...] = _dot(de_ref[...], webot_ref[...])
    tb_o[...] = _dot(emb, b_ref[...]) + bb_ref[...]
    td_o[...] = _dot(emb, d_ref[...]) + db_ref[...]
    te_o[...] = _dot(emb, e_ref[...]) + eb_ref[...]
    ta = _dot(emb, a_ref[...]) + ab_ref[...]
    nt_o[...] = jnp.concatenate([emb, ta], axis=1)


def _tab0(emb_h, mp, lp0):
    full = pl.BlockSpec(None, lambda: tuple())
    outs = (
        jax.ShapeDtypeStruct((V, D * D), F32),
        jax.ShapeDtypeStruct((V, D), F32),
        jax.ShapeDtypeStruct((V, D), F32),
        jax.ShapeDtypeStruct((V, D), F32),
        jax.ShapeDtypeStruct((V, D), F32),
        jax.ShapeDtypeStruct((V, D), F32),
        jax.ShapeDtypeStruct((V, D), F32),
        jax.ShapeDtypeStruct((V, 2 * D), F32),
    )
    args = (emb_h, mp['W1'], mp['b1'].reshape(1, -1), mp['W2'],
            mp['b2'].reshape(1, -1), mp['W_ep'][:D], mp['W_ep'][D:],
            mp['src_emb'], mp['dst_emb'],
            lp0['A'], lp0['Ab'].reshape(1, -1), lp0['B'],
            lp0['Bb'].reshape(1, -1), lp0['D'], lp0['Db'].reshape(1, -1),
            lp0['E'], lp0['Eb'].reshape(1, -1))
    return pl.pallas_call(
        _tab0_body,
        out_shape=outs,
    )(*args)


_CB = 2000


def _count_body(hs_ref, hd_ref, cnt_o, acc):
    i = pl.program_id(0)

    @pl.when(i == 0)
    def _():
        acc[...] = jnp.zeros_like(acc)

    lanes = lax.broadcasted_iota(jnp.int32, (_CB, V), 1)
    ohs = (hs_ref[...] == lanes).astype(jnp.bfloat16)
    ohd = (hd_ref[...] == lanes).astype(jnp.bfloat16)
    acc[...] += lax.dot_general(ohs, ohd, (((0,), (0,)), ((), ())),
                                preferred_element_type=F32)

    @pl.when(i == pl.num_programs(0) - 1)
    def _():
        cnt_o[...] = acc[...]


def _count(hs_col, hd_col):
    grid = E // _CB
    return pl.pallas_call(
        _count_body,
        grid=(grid,),
        in_specs=[pl.BlockSpec((_CB, 1), lambda i: (i, 0)),
                  pl.BlockSpec((_CB, 1), lambda i: (i, 0))],
        out_specs=pl.BlockSpec((V, V), lambda i: (0, 0)),
        out_shape=jax.ShapeDtypeStruct((V, V), F32),
        scratch_shapes=[pltpu.VMEM((V, V), F32)],
    )(hs_col, hd_col)


def _tab1_body(p1_ref, p2_ref, w3_ref, b3_ref, l1_ref, l2_ref, bep_ref,
               cnt_ref, tx_o, s1_o, s2_o, s1a, s2a):
    u = pl.program_id(0)

    @pl.when(u == 0)
    def _():
        s1a[...] = jnp.zeros_like(s1a)
        s2a[...] = jnp.zeros_like(s2a)

    p1u = p1_ref[0]
    tu = lax.dot_general(p2_ref[...], p1u, (((1,), (1,)), ((), ())),
                         preferred_element_type=F32)
    tx = (_dot(tu, w3_ref[...]) + b3_ref[...] + l1_ref[0]
          + l2_ref[...] + bep_ref[...])
    tx_o[...] = tx
    w = cnt_ref[0] * (1.0 / E)
    s1a[...] += _dot(w, tx)
    s2a[...] += _dot(w, tx * tx)

    @pl.when(u == pl.num_programs(0) - 1)
    def _():
        s1_o[...] = s1a[...]
        s2_o[...] = s2a[...]


def _tab1(p1v3, p2v, mp, l1t, l2t, cnt):
    return pl.pallas_call(
        _tab1_body,
        grid=(V,),
        in_specs=[
            pl.BlockSpec((1, D, D), lambda u: (u, 0, 0)),
            pl.BlockSpec((V, D), lambda u: (0, 0)),
            pl.BlockSpec((D, D), lambda u: (0, 0)),
            pl.BlockSpec((1, D), lambda u: (0, 0)),
            pl.BlockSpec((1, 1, D), lambda u: (u, 0, 0)),
            pl.BlockSpec((V, D), lambda u: (0, 0)),
            pl.BlockSpec((1, D), lambda u: (0, 0)),
            pl.BlockSpec((1, 1, V), lambda u: (u, 0, 0)),
        ],
        out_specs=[
            pl.BlockSpec((V, D), lambda u: (u, 0)),
            pl.BlockSpec((1, D), lambda u: (0, 0)),
            pl.BlockSpec((1, D), lambda u: (0, 0)),
        ],
        out_shape=[
            jax.ShapeDtypeStruct((V * V, D), F32),
            jax.ShapeDtypeStruct((1, D), F32),
            jax.ShapeDtypeStruct((1, D), F32),
        ],
        scratch_shapes=[pltpu.VMEM((1, D), F32), pltpu.VMEM((1, D), F32)],
    )(p1v3, p2v, mp['W_ep3'], mp['b_ep3'].reshape(1, -1),
      l1t.reshape(V, 1, D), l2t, mp['b_ep'].reshape(1, -1),
      cnt.reshape(V, 1, V))


def _tab2_body(tx_ref, s1_ref, s2_ref, g_ref, b_ref, td_ref, te_ref, tb_ref,
               c_ref, cb_ref, cnt_ref,
               tee_o, teh_o, tsg_o, s1e_o, s2e_o, s1a, s2a):
    u = pl.program_id(0)

    @pl.when(u == 0)
    def _():
        s1a[...] = jnp.zeros_like(s1a)
        s2a[...] = jnp.zeros_like(s2a)

    mu = s1_ref[...]
    inv = lax.rsqrt(s2_ref[...] - mu * mu + EPS)
    tee = jnp.maximum((tx_ref[...] - mu) * inv * g_ref[...] + b_ref[...], 0.0)
    tee_o[...] = tee
    teh = _dot(tee, c_ref[...]) + cb_ref[...] + td_ref[...] + te_ref[0]
    teh_o[...] = teh
    tsig = jax.nn.sigmoid(teh)
    tsg_o[...] = jnp.concatenate([tsig, tsig * tb_ref[0]], axis=1)
    w = cnt_ref[0] * (1.0 / E)
    s1a[...] += _dot(w, teh)
    s2a[...] += _dot(w, teh * teh)

    @pl.when(u == pl.num_programs(0) - 1)
    def _():
        s1e_o[...] = s1a[...]
        s2e_o[...] = s2a[...]


def _tab2(tx, s1m, s2m, mp, lp0, tbt, tdt, tet, cnt):
    return pl.pallas_call(
        _tab2_body,
        grid=(V,),
        in_specs=[
            pl.BlockSpec((V, D), lambda u: (u, 0)),
            pl.BlockSpec((1, D), lambda u: (0, 0)),
            pl.BlockSpec((1, D), lambda u: (0, 0)),
            pl.BlockSpec((1, D), lambda u: (0, 0)),
            pl.BlockSpec((1, D), lambda u: (0, 0)),
            pl.BlockSpec((V, D), lambda u: (0, 0)),
            pl.BlockSpec((1, 1, D), lambda u: (u, 0, 0)),
            pl.BlockSpec((1, 1, D), lambda u: (u, 0, 0)),
            pl.BlockSpec((D, D), lambda u: (0, 0)),
            pl.BlockSpec((1, D), lambda u: (0, 0)),
            pl.BlockSpec((1, 1, V), lambda u: (u, 0, 0)),
        ],
        out_specs=[
            pl.BlockSpec((V, D), lambda u: (u, 0)),
            pl.BlockSpec((V, D), lambda u: (u, 0)),
            pl.BlockSpec((V, 2 * D), lambda u: (u, 0)),
            pl.BlockSpec((1, D), lambda u: (0, 0)),
            pl.BlockSpec((1, D), lambda u: (0, 0)),
        ],
        out_shape=[
            jax.ShapeDtypeStruct((V * V, D), F32),
            jax.ShapeDtypeStruct((V * V, D), F32),
            jax.ShapeDtypeStruct((V * V, 2 * D), F32),
            jax.ShapeDtypeStruct((1, D), F32),
            jax.ShapeDtypeStruct((1, D), F32),
        ],
        scratch_shapes=[pltpu.VMEM((1, D), F32), pltpu.VMEM((1, D), F32)],
    )(tx, s1m, s2m, mp['bn_g'].reshape(1, -1), mp['bn_b'].reshape(1, -1),
      tdt, tet.reshape(V, 1, D), tbt.reshape(V, 1, D), lp0['C'],
      lp0['Cb'].reshape(1, -1), cnt.reshape(V, 1, V))


def _tab3_body(tee_ref, teh_ref, s1_ref, s2_ref, g_ref, b_ref, c_ref, cb_ref,
               tq_o):
    mu = s1_ref[...]
    inv = lax.rsqrt(s2_ref[...] - mu * mu + EPS)
    te2 = tee_ref[...] + jnp.maximum(
        (teh_ref[...] - mu) * inv * g_ref[...] + b_ref[...], 0.0)
    tq_o[...] = jnp.concatenate(
        [_dot(te2, c_ref[...]) + cb_ref[...], te2], axis=1)


def _tab3(tee, teh, s1e, s2e, lp0, lp1):
    return pl.pallas_call(
        _tab3_body,
        grid=(V,),
        in_specs=[
            pl.BlockSpec((V, D), lambda u: (u, 0)),
            pl.BlockSpec((V, D), lambda u: (u, 0)),
            pl.BlockSpec((1, D), lambda u: (0, 0)),
            pl.BlockSpec((1, D), lambda u: (0, 0)),
            pl.BlockSpec((1, D), lambda u: (0, 0)),
            pl.BlockSpec((1, D), lambda u: (0, 0)),
            pl.BlockSpec((D, D), lambda u: (0, 0)),
            pl.BlockSpec((1, D), lambda u: (0, 0)),
        ],
        out_specs=pl.BlockSpec((V, 2 * D), lambda u: (u, 0)),
        out_shape=jax.ShapeDtypeStruct((V * V, 2 * D), F32),
    )(tee, teh, s1e, s2e, lp0['bne_g'].reshape(1, -1),
      lp0['bne_b'].reshape(1, -1), lp1['C'], lp1['Cb'].reshape(1, -1))


_NB = 2048


def _row_mask(nrows, limit):
    rows = (pl.program_id(0) * nrows
            + lax.broadcasted_iota(jnp.int32, (nrows, 1), 0))
    return rows < limit


def _mm_body(hh_ref, a_ref, ab_ref, b_ref, bb_ref, d_ref, db_ref,
             e_ref, eb_ref, ah_o, dh_o, ebo_o):
    m = _row_mask(_NB, N)
    hh = hh_ref[...]
    ah_o[...] = jnp.where(m, _dot(hh, a_ref[...]) + ab_ref[...], 0.0)
    dh = jnp.where(m, _dot(hh, d_ref[...]) + db_ref[...], 0.0)
    dh_o[...] = jnp.concatenate([dh, jnp.zeros_like(dh)], axis=1)
    eh = _dot(hh, e_ref[...]) + eb_ref[...]
    bh = _dot(hh, b_ref[...]) + bb_ref[...]
    ebo_o[...] = jnp.where(m, jnp.concatenate([eh, bh], axis=1), 0.0)


def _tc_mm(hh, lp):
    return pl.pallas_call(
        _mm_body,
        grid=(N_PAD // _NB,),
        in_specs=[pl.BlockSpec((_NB, D), lambda i: (i, 0))]
        + [pl.BlockSpec((D, D), lambda i: (0, 0)),
           pl.BlockSpec((1, D), lambda i: (0, 0))] * 4,
        out_specs=[
            pl.BlockSpec((_NB, D), lambda i: (i, 0)),
            pl.BlockSpec((_NB, 2 * D), lambda i: (i, 0)),
            pl.BlockSpec((_NB, 2 * D), lambda i: (i, 0)),
        ],
        out_shape=[
            jax.ShapeDtypeStruct((N_PAD, D), F32),
            jax.ShapeDtypeStruct((N_PAD, 2 * D), F32),
            jax.ShapeDtypeStruct((N_PAD, 2 * D), F32),
        ],
    )(hh, lp['A'], lp['Ab'].reshape(1, -1), lp['B'], lp['Bb'].reshape(1, -1),
      lp['D'], lp['Db'].reshape(1, -1), lp['E'], lp['Eb'].reshape(1, -1))


def _ha_body(ah_ref, dn_ref, hn_o, s1_o, s2_o, s1a, s2a):
    i = pl.program_id(0)

    @pl.when(i == 0)
    def _():
        s1a[...] = jnp.zeros_like(s1a)
        s2a[...] = jnp.zeros_like(s2a)

    dn = dn_ref[0] + dn_ref[1]
    den = dn[:, :D]
    num = dn[:, D:]
    hn = ah_ref[...] + num / (den + 1e-6)
    hn = jnp.where(_row_mask(_NB, N), hn, 0.0)
    hn_o[...] = hn
    s1a[...] += jnp.sum(hn, axis=0, keepdims=True)
    s2a[...] += jnp.sum(hn * hn, axis=0, keepdims=True)

    @pl.when(i == pl.num_programs(0) - 1)
    def _():
        s1_o[...] = s1a[...]
        s2_o[...] = s2a[...]


def _tc_ha(ah, dn_p):
    return pl.pallas_call(
        _ha_body,
        grid=(N_PAD // _NB,),
        in_specs=[
            pl.BlockSpec((_NB, D), lambda i: (i, 0)),
            pl.BlockSpec((2, _NB, 2 * D), lambda i: (0, i, 0)),
        ],
        out_specs=[
            pl.BlockSpec((_NB, D), lambda i: (i, 0)),
            pl.BlockSpec((1, D), lambda i: (0, 0)),
            pl.BlockSpec((1, D), lambda i: (0, 0)),
        ],
        out_shape=[
            jax.ShapeDtypeStruct((N_PAD, D), F32),
            jax.ShapeDtypeStruct((1, D), F32),
            jax.ShapeDtypeStruct((1, D), F32),
        ],
        scratch_shapes=[pltpu.VMEM((1, D), F32), pltpu.VMEM((1, D), F32)],
    )(ah, dn_p)


def _hb_body(hh_ref, hn_ref, s1_ref, s2_ref, g_ref, b_ref, hho_o):
    mu = s1_ref[...] * (1.0 / N)
    var = s2_ref[...] * (1.0 / N) - mu * mu
    inv = lax.rsqrt(var + EPS)
    hho = hh_ref[...] + jnp.maximum(
        (hn_ref[...] - mu) * inv * g_ref[...] + b_ref[...], 0.0)
    hho_o[...] = jnp.where(_row_mask(_NB, N), hho, 0.0)


def _tc_hb(hh, hn, s1, s2, lp):
    return pl.pallas_call(
        _hb_body,
        grid=(N_PAD // _NB,),
        in_specs=[
            pl.BlockSpec((_NB, D), lambda i: (i, 0)),
            pl.BlockSpec((_NB, D), lambda i: (i, 0)),
            pl.BlockSpec((1, D), lambda i: (0, 0)),
            pl.BlockSpec((1, D), lambda i: (0, 0)),
            pl.BlockSpec((1, D), lambda i: (0, 0)),
            pl.BlockSpec((1, D), lambda i: (0, 0)),
        ],
        out_specs=pl.BlockSpec((_NB, D), lambda i: (i, 0)),
        out_shape=jax.ShapeDtypeStruct((N_PAD, D), F32),
    )(hh, hn, s1, s2, lp['bnh_g'].reshape(1, -1), lp['bnh_b'].reshape(1, -1))


_EB = 2048


def _make_e_body(with_ce):
    def body(*refs):
        if with_ce:
            (eh_ref, ee_ref, ss_ref, g_ref, b_ref, c_ref, cb_ref,
             eo_o, ce_o) = refs
        else:
            (eh_ref, ee_ref, ss_ref, g_ref, b_ref, eo_o) = refs
        ss = jnp.sum(ss_ref[...], axis=0, keepdims=True) * (1.0 / E)
        mu = ss[:, :D]
        inv = lax.rsqrt(ss[:, D:] - mu * mu + EPS)
        mu2 = jnp.concatenate([mu, mu], axis=1)
        inv2 = jnp.concatenate([inv, inv], axis=1)
        g2 = jnp.concatenate([g_ref[...], g_ref[...]], axis=1)
        b2 = jnp.concatenate([b_ref[...], b_ref[...]], axis=1)
        eo = ee_ref[...] + jnp.maximum(
            (eh_ref[...] - mu2) * inv2 * g2 + b2, 0.0)
        m = _row_mask(_EB, E // 2)
        eo = jnp.where(m, eo, 0.0)
        eo_o[...] = eo
        if with_ce:
            ce_o[...] = _dot(eo, c_ref[...]) + cb_ref[...]
    return body


def _tc_e(ehat, eein, ss, lp, lp_next):
    with_ce = lp_next is not None
    in_specs = [
        pl.BlockSpec((_EB, 2 * D), lambda i: (i, 0)),
        pl.BlockSpec((_EB, 2 * D), lambda i: (i, 0)),
        pl.BlockSpec((NW, 2 * D), lambda i: (0, 0)),
        pl.BlockSpec((1, D), lambda i: (0, 0)),
        pl.BlockSpec((1, D), lambda i: (0, 0)),
    ]
    args = [ehat, eein, ss, lp['bne_g'].reshape(1, -1),
            lp['bne_b'].reshape(1, -1)]
    out_specs = [pl.BlockSpec((_EB, 2 * D), lambda i: (i, 0))]
    out_shape = [jax.ShapeDtypeStruct((E_HALF, 2 * D), F32)]
    if with_ce:
        zdd = jnp.zeros((D, D), F32)
        c2 = jnp.block([[lp_next['C'], zdd], [zdd, lp_next['C']]])
        cb2 = jnp.tile(lp_next['Cb'], 2).reshape(1, -1)
        in_specs += [pl.BlockSpec((2 * D, 2 * D), lambda i: (0, 0)),
                     pl.BlockSpec((1, 2 * D), lambda i: (0, 0))]
        args += [c2, cb2]
        out_specs += [pl.BlockSpec((_EB, 2 * D), lambda i: (i, 0))]
        out_shape += [jax.ShapeDtypeStruct((E_HALF, 2 * D), F32)]
    res = pl.pallas_call(
        _make_e_body(with_ce),
        grid=(E_HALF // _EB,),
        in_specs=in_specs,
        out_specs=out_specs,
        out_shape=out_shape,
    )(*args)
    return res if with_ce else (res[0], None)


def _mlp_body(hh_ref, w0_ref, b0_ref, w1_ref, b1_ref, w2_ref, b2_ref, y_o):
    y = jnp.maximum(_dot(hh_ref[...], w0_ref[...]) + b0_ref[...], 0.0)
    y = jnp.maximum(_dot(y, w1_ref[...]) + b1_ref[...], 0.0)
    y_o[...] = _dot(y, w2_ref[...]) + b2_ref[...]


def _tc_mlp(hh, mlp):
    d0 = mlp['W0'].shape[1]
    d1 = mlp['W1'].shape[1]
    d2 = mlp['W2'].shape[1]
    return pl.pallas_call(
        _mlp_body,
        grid=(N_PAD // _NB,),
        in_specs=[
            pl.BlockSpec((_NB, D), lambda i: (i, 0)),
            pl.BlockSpec((D, d0), lambda i: (0, 0)),
            pl.BlockSpec((1, d0), lambda i: (0, 0)),
            pl.BlockSpec((d0, d1), lambda i: (0, 0)),
            pl.BlockSpec((1, d1), lambda i: (0, 0)),
            pl.BlockSpec((d1, d2), lambda i: (0, 0)),
            pl.BlockSpec((1, d2), lambda i: (0, 0)),
        ],
        out_specs=pl.BlockSpec((_NB, d2), lambda i: (i, 0)),
        out_shape=jax.ShapeDtypeStruct((N_PAD, d2), F32),
    )(hh, mlp['W0'], mlp['b0'].reshape(1, -1), mlp['W1'],
      mlp['b1'].reshape(1, -1), mlp['W2'], mlp['b2'].reshape(1, -1))


def kernel(h, edge_index, e, emb_h, gtp, merg, layers, mlp):
    del e, gtp
    h = h.astype(jnp.int32)
    src = edge_index[0].astype(jnp.int32)
    dst = edge_index[1].astype(jnp.int32)

    h_pad = jnp.concatenate([h, jnp.zeros((N_PAD - N,), jnp.int32)])
    pad_e = jnp.full((E_PAD - E,), DUMMY, jnp.int32)
    src2 = jnp.concatenate([src, pad_e]).reshape(NW, EPW)
    dst2 = jnp.concatenate([dst, pad_e]).reshape(NW, EPW)
    zrow = jnp.zeros((TPS, 2 * D), F32)

    lp = layers

    p1v, p2v, l1t, l2t, tbt, tdt, tet, nt = _tab0(emb_h, merg, lp[0])
    src3 = src2.reshape(NW, NCH, CH)
    dst3 = dst2.reshape(NW, NCH, CH)
    pid2, hs2, hd2, ntg = _sc0(h_pad, src3, dst3, nt)
    pid3 = pid2.reshape(NW, NCH, CH)
    hs_col = hs2.reshape(E_PAD, 1)[:E]
    hd_col = hd2.reshape(E_PAD, 1)[:E]

    cnt = _count(hs_col, hd_col)
    tx, s1m, s2m = _tab1(p1v.reshape(V, D, D), p2v, merg, l1t, l2t, cnt)
    tee, teh, tsg, s1e1, s2e1 = _tab2(
        tx, s1m, s2m, merg, lp[0], tbt, tdt, tet, cnt)
    tq = _tab3(tee, teh, s1e1, s2e1, lp[0], lp[1])

    dn_p = _sc_l1(pid3, dst3, tsg, zrow)
    hemb = ntg[:, :D]
    ah1 = ntg[:, D:]
    hn, s1h, s2h = _tc_ha(ah1, dn_p)
    hh = _tc_hb(hemb, hn, s1h, s2h, lp[0])

    sc_l2 = _make_sc_layer(gather_ce=True, with_stats=True)
    sc_l3 = _make_sc_layer(gather_ce=False, with_stats=True)
    sc_l4 = _make_sc_layer(gather_ce=False, with_stats=False)

    ah, dh, ebt = _tc_mm(hh, lp[1])
    dn_p, ehat, ss, eein = sc_l2(src3, dst3, pid3, tq, dh, ebt, zrow)
    hn, s1h, s2h = _tc_ha(ah, dn_p)
    hh = _tc_hb(hh, hn, s1h, s2h, lp[1])
    ee, ce = _tc_e(ehat, eein, ss, lp[1], lp[2])

    ah, dh, ebt = _tc_mm(hh, lp[2])
    dn_p, ehat, ss = sc_l3(src3, dst3, ce, dh, ebt, zrow)
    hn, s1h, s2h = _tc_ha(ah, dn_p)
    hh = _tc_hb(hh, hn, s1h, s2h, lp[2])
    ee, ce = _tc_e(ehat, ee, ss, lp[2], lp[3])

    ah, dh, ebt = _tc_mm(hh, lp[3])
    dn_p, = sc_l4(src3, dst3, ce, dh, ebt, zrow)
    hn, s1h, s2h = _tc_ha(ah, dn_p)
    hh = _tc_hb(hh, hn, s1h, s2h, lp[3])

    return _tc_mlp(hh, mlp)[:N]


# speedup vs baseline: 5.7525x; 1.0203x over previous
import functools

import jax
import jax.numpy as jnp
from jax import lax
from jax.experimental import pallas as pl
from jax.experimental.pallas import tpu as pltpu
from jax.experimental.pallas import tpu_sc as plsc

F32 = jnp.float32
EPS = 1e-5

N = 10000
E = 320000
D = 64
V = 128
NW = 32
N_PAD = 10240
E_PAD = 327680
EPW = E_PAD // NW
NCH = 80
CH = 128
NPW = N_PAD // NW
TPS = N_PAD // 16
DUMMY = N

_SC_MESH = dict(core_axis_name="c", subcore_axis_name="s")

CHL = 32
NRB = 4
CPB = NRB * (CH // CHL)
NBLKL = NCH // NRB
NCHL = EPW // CHL
E_HALF = E_PAD // 2


def _wid():
    return lax.axis_index("c") * 16 + lax.axis_index("s")


def _sc0_body(h_hbm, src_hbm, dst_hbm, nt_hbm,
              pid_o, hs_o, hd_o, ntg_o,
              hv, sv, dv, pv, hsv, hdv, bufs, bufd, bufn, sema, semb, sem):
    wid = _wid()
    pltpu.sync_copy(h_hbm, hv)
    pltpu.sync_copy(src_hbm.at[wid], sv)
    pltpu.sync_copy(dst_hbm.at[wid], dv)

    def chunk(j, _):
        ga = pltpu.async_copy(h_hbm.at[sv.at[j]], bufs, sema)
        gb = pltpu.async_copy(h_hbm.at[dv.at[j]], bufd, semb)
        ga.wait()
        gb.wait()

        def step(k, _2):
            sl = pl.ds(k * 16, 16)
            hs = bufs[sl]
            hd = bufd[sl]
            o = pl.ds(j * CH + k * 16, 16)
            pv[o] = hs * V + hd
            hsv[o] = hs
            hdv[o] = hd
            return 0

        lax.fori_loop(0, CH // 16, step, 0)
        return 0

    lax.fori_loop(0, NCH, chunk, 0)
    pltpu.sync_copy(pv, pid_o.at[wid])
    pltpu.sync_copy(hsv, hs_o.at[wid])
    pltpu.sync_copy(hdv, hd_o.at[wid])

    nb = wid * NPW
    for off, ln in ((0, 128), (128, 128), (256, 64)):
        pltpu.async_copy(nt_hbm.at[hv.at[pl.ds(nb + off, ln)]],
                         bufn.at[pl.ds(0, ln)], sem).wait()
        pltpu.sync_copy(bufn.at[pl.ds(0, ln)],
                        ntg_o.at[pl.ds(nb + off, ln), :])


def _sc0(h_pad, src3, dst3, nt):
    fn = pl.kernel(
        _sc0_body,
        out_type=(
            jax.ShapeDtypeStruct((NW, EPW), jnp.int32),
            jax.ShapeDtypeStruct((NW, EPW), jnp.int32),
            jax.ShapeDtypeStruct((NW, EPW), jnp.int32),
            jax.ShapeDtypeStruct((N_PAD, 2 * D), F32),
        ),
        mesh=plsc.VectorSubcoreMesh(**_SC_MESH),
        scratch_types=(
            pltpu.VMEM((N_PAD,), jnp.int32),
            pltpu.VMEM((NCH, CH), jnp.int32),
            pltpu.VMEM((NCH, CH), jnp.int32),
            pltpu.VMEM((EPW,), jnp.int32),
            pltpu.VMEM((EPW,), jnp.int32),
            pltpu.VMEM((EPW,), jnp.int32),
            pltpu.VMEM((CH,), jnp.int32),
            pltpu.VMEM((CH,), jnp.int32),
            pltpu.VMEM((128, 2 * D), F32),
            pltpu.SemaphoreType.DMA,
            pltpu.SemaphoreType.DMA,
            pltpu.SemaphoreType.DMA,
        ),
    )
    return fn(h_pad, src3, dst3, nt)


def _sc_l1_body(pid_hbm, dst_hbm, tsg_hbm, z_hbm,
                dn_o,
                pv, dv, buf0, buf1, dn_s, semg0, semg1, semo0, semo1):
    cid = lax.axis_index("c")
    sid = lax.axis_index("s")
    wid = cid * 16 + sid
    tile_rows = pl.ds(sid * TPS, TPS)
    pltpu.sync_copy(z_hbm, dn_s.at[tile_rows, :])
    plsc.subcore_barrier()

    bufs = (buf0, buf1)
    sems = (semg0, semg1)
    semo = (semo0, semo1)

    def issue(k, g, h, s):
        @pl.when(k >= 2)
        def _():
            pltpu.make_async_copy(
                bufs[s], dn_s.at[dv.at[0, pl.ds(0, 64)]], semo[s]).wait()

        pltpu.async_copy(tsg_hbm.at[pv.at[g, pl.ds(h * 64, 64)]],
                         bufs[s], sems[s])

    def proc(g, h, s):
        pltpu.make_async_copy(tsg_hbm.at[pv.at[0, pl.ds(0, 64)]],
                              bufs[s], sems[s]).wait()
        pltpu.async_copy(bufs[s], dn_s.at[dv.at[g, pl.ds(h * 64, 64)]],
                         semo[s], add=True)

    def blk(b, _):
        pltpu.sync_copy(pid_hbm.at[wid, pl.ds(b * 8, 8)], pv)
        pltpu.sync_copy(dst_hbm.at[wid, pl.ds(b * 8, 8)], dv)
        issue(b * 16, 0, 0, 0)

        def pair(g, _2):
            issue(b * 16 + 2 * g + 1, g, 1, 1)
            proc(g, 0, 0)

            @pl.when(g < 7)
            def _():
                issue(b * 16 + 2 * g + 2, g + 1, 0, 0)

            proc(g, 1, 1)
            return 0

        lax.fori_loop(0, 8, pair, 0)
        return 0

    lax.fori_loop(0, NCH // 8, blk, 0)
    for s in (0, 1):
        pltpu.make_async_copy(
            bufs[s], dn_s.at[dv.at[0, pl.ds(0, 64)]], semo[s]).wait()
    plsc.subcore_barrier()
    pltpu.sync_copy(dn_s.at[tile_rows, :], dn_o.at[cid, tile_rows, :])


def _sc_l1(pid3, dst3, tsg, zrow):
    fn = pl.kernel(
        _sc_l1_body,
        out_type=(
            jax.ShapeDtypeStruct((2, N_PAD, 2 * D), F32),
        ),
        mesh=plsc.VectorSubcoreMesh(**_SC_MESH),
        scratch_types=(
            pltpu.VMEM((8, CH), jnp.int32),
            pltpu.VMEM((8, CH), jnp.int32),
            pltpu.VMEM((64, 2 * D), F32),
            pltpu.VMEM((64, 2 * D), F32),
            pltpu.VMEM_SHARED((N_PAD, 2 * D), F32),
            pltpu.SemaphoreType.DMA,
            pltpu.SemaphoreType.DMA,
            pltpu.SemaphoreType.DMA,
            pltpu.SemaphoreType.DMA,
        ),
    )
    return fn(pid3, dst3, tsg, zrow)[0]


def _make_sc_layer(gather_ce, with_stats):
    def body(*refs):
        it = iter(refs)
        src_hbm = next(it)
        dst_hbm = next(it)
        if gather_ce:
            pid_hbm = next(it)
            tq_hbm = next(it)
        else:
            ce_hbm = next(it)
        dh_hbm = next(it)
        eb_hbm = next(it)
        z_hbm = next(it)
        dn_o = next(it)
        if with_stats:
            ehat_o = next(it)
            ss_o = next(it)
        if gather_ce:
            eein_o = next(it)
        sv = next(it)
        dv = next(it)
        if gather_ce:
            pv = next(it)
            bufi = (next(it), next(it))
        else:
            bufc = (next(it), next(it))
        bufd = (next(it), next(it))
        bufeb = (next(it), next(it))
        bufo = (next(it), next(it))
        bufe = (next(it), next(it))
        if gather_ce:
            bufw = (next(it), next(it))
        bufs = next(it)
        dn_s = next(it)
        semd = (next(it), next(it))
        seme = (next(it), next(it))
        semc = (next(it), next(it))
        semo = (next(it), next(it))
        semw = (next(it), next(it))

        cid = lax.axis_index("c")
        sid = lax.axis_index("s")
        wid = cid * 16 + sid
        base = wid * EPW
        tile_rows = pl.ds(sid * TPS, TPS)
        pltpu.sync_copy(z_hbm, dn_s.at[tile_rows, :])
        plsc.subcore_barrier()
        nreal = jnp.where(wid == NW - 1, (E - (NW - 1) * EPW) // CHL, NCHL)

        zv = jnp.zeros((16,), F32)

        def issue(b, g, s):
            row = g // 4
            col = (g % 4) * CHL
            pltpu.async_copy(dh_hbm.at[dv.at[row, pl.ds(col, CHL)]],
                             bufd[s], semd[s])
            pltpu.async_copy(eb_hbm.at[sv.at[row, pl.ds(col, CHL)]],
                             bufeb[s], seme[s])
            if gather_ce:
                pltpu.async_copy(tq_hbm.at[pv.at[row, pl.ds(col, CHL)]],
                                 bufi[s], semc[s])
            else:
                off = base + (b * CPB + g) * CHL
                hoff = pl.multiple_of(off // 2, CHL // 2)
                pltpu.async_copy(ce_hbm.at[pl.ds(hoff, CHL // 2), :],
                                 bufc[s], semc[s])

        def wait_set(s):
            pltpu.make_async_copy(
                dh_hbm.at[dv.at[0, pl.ds(0, CHL)]], bufd[s], semd[s]).wait()
            pltpu.make_async_copy(
                eb_hbm.at[sv.at[0, pl.ds(0, CHL)]], bufeb[s], seme[s]).wait()
            if gather_ce:
                pltpu.make_async_copy(
                    tq_hbm.at[pv.at[0, pl.ds(0, CHL)]], bufi[s],
                    semc[s]).wait()
            else:
                pltpu.make_async_copy(
                    ce_hbm.at[pl.ds(0, CHL // 2), :], bufc[s],
                    semc[s]).wait()

        def work(b, g, s, carry):
            row = g // 4
            col = (g % 4) * CHL
            di = dv.at[row, pl.ds(col, CHL)]
            off = base + (b * CPB + g) * CHL
            hoff = pl.multiple_of(off // 2, CHL // 2)
            bd, be_, bo = bufd[s], bufeb[s], bufo[s]

            @pl.when(b * CPB + g >= 2)
            def _():
                pltpu.make_async_copy(
                    bo, dn_s.at[dv.at[0, pl.ds(0, CHL)]], semo[s]).wait()
                if with_stats:
                    pltpu.make_async_copy(
                        bufe[s], ehat_o.at[pl.ds(0, CHL // 2), :],
                        semw[s]).wait()
                if gather_ce:
                    pltpu.make_async_copy(
                        bufw[s], eein_o.at[pl.ds(0, CHL // 2), :],
                        semw[s]).wait()
            if gather_ce:
                bi = bufi[s]
            else:
                bc = bufc[s]

            def prow(p, rc):
                out = list(rc)
                for edge in range(2):
                    r = 2 * p + edge
                    for c in range(4):
                        sl = pl.ds(c * 16, 16)
                        sl2 = pl.ds(D + c * 16, 16)
                        pcol = pl.ds(edge * D + c * 16, 16)
                        if gather_ce:
                            ce_v = bi[r, sl]
                        else:
                            ce_v = bc[p, pcol]
                        eh = bd[r, sl] + be_[r, sl] + ce_v
                        bh = be_[r, sl2]
                        sg = 1.0 / (1.0 + jnp.exp(-eh))
                        bo[r, sl] = sg
                        bo[r, sl2] = sg * bh
                        if with_stats:
                            bufe[s][p, pcol] = eh
                            out[c] = out[c] + eh
                            out[4 + c] = out[4 + c] + eh * eh
                        if gather_ce:
                            bufw[s][p, pcol] = bi[r, sl2]
                return tuple(out)

            csum = lax.fori_loop(0, CHL // 2, prow, (zv,) * 8)
            pltpu.async_copy(bo, dn_s.at[di], semo[s], add=True)
            if with_stats:
                pltpu.async_copy(bufe[s],
                                 ehat_o.at[pl.ds(hoff, CHL // 2), :], semw[s])
            if gather_ce:
                pltpu.async_copy(bufw[s],
                                 eein_o.at[pl.ds(hoff, CHL // 2), :], semw[s])
            m = jnp.where(b * CPB + g < nreal, 1.0, 0.0).astype(F32)
            return tuple(carry[i] + m * csum[i] for i in range(8))

        def blk(b, bcarry):
            pltpu.sync_copy(src_hbm.at[wid, pl.ds(b * NRB, NRB)], sv)
            pltpu.sync_copy(dst_hbm.at[wid, pl.ds(b * NRB, NRB)], dv)
            if gather_ce:
                pltpu.sync_copy(pid_hbm.at[wid, pl.ds(b * NRB, NRB)], pv)
            issue(b, 0, 0)

            def pairs(gg, carry):
                issue(b, 2 * gg + 1, 1)
                wait_set(0)
                carry = work(b, 2 * gg, 0, carry)

                @pl.when(gg < CPB // 2 - 1)
                def _():
                    issue(b, 2 * gg + 2, 0)

                wait_set(1)
                return work(b, 2 * gg + 1, 1, carry)

            return lax.fori_loop(0, CPB // 2, pairs, bcarry)

        stats = lax.fori_loop(0, NBLKL, blk, (zv,) * 8)
        for s in (0, 1):
            pltpu.make_async_copy(
                bufo[s], dn_s.at[dv.at[0, pl.ds(0, CHL)]], semo[s]).wait()
            if with_stats:
                pltpu.make_async_copy(
                    bufe[s], ehat_o.at[pl.ds(0, CHL // 2), :], semw[s]).wait()
            if gather_ce:
                pltpu.make_async_copy(
                    bufw[s], eein_o.at[pl.ds(0, CHL // 2), :], semw[s]).wait()
        if with_stats:
            for c in range(4):
                bufs[pl.ds(c * 16, 16)] = stats[c]
                bufs[pl.ds(D + c * 16, 16)] = stats[4 + c]
            pltpu.sync_copy(bufs, ss_o.at[wid])
        plsc.subcore_barrier()
        pltpu.sync_copy(dn_s.at[tile_rows, :], dn_o.at[cid, tile_rows, :])

    out_type = [
        jax.ShapeDtypeStruct((2, N_PAD, 2 * D), F32),
    ]
    if with_stats:
        out_type += [
            jax.ShapeDtypeStruct((E_HALF, 2 * D), F32),
            jax.ShapeDtypeStruct((NW, 2 * D), F32),
        ]
    if gather_ce:
        out_type += [jax.ShapeDtypeStruct((E_HALF, 2 * D), F32)]

    scratch = [
        pltpu.VMEM((NRB, CH), jnp.int32),
        pltpu.VMEM((NRB, CH), jnp.int32),
    ]
    if gather_ce:
        scratch += [pltpu.VMEM((NRB, CH), jnp.int32),
                    pltpu.VMEM((CHL, 2 * D), F32),
                    pltpu.VMEM((CHL, 2 * D), F32)]
    else:
        scratch += [pltpu.VMEM((CHL // 2, 2 * D), F32),
                    pltpu.VMEM((CHL // 2, 2 * D), F32)]
    scratch += [
        pltpu.VMEM((CHL, 2 * D), F32),
        pltpu.VMEM((CHL, 2 * D), F32),
        pltpu.VMEM((CHL, 2 * D), F32),
        pltpu.VMEM((CHL, 2 * D), F32),
        pltpu.VMEM((CHL, 2 * D), F32),
        pltpu.VMEM((CHL, 2 * D), F32),
        pltpu.VMEM((CHL // 2, 2 * D), F32),
        pltpu.VMEM((CHL // 2, 2 * D), F32),
    ]
    if gather_ce:
        scratch += [pltpu.VMEM((CHL // 2, 2 * D), F32),
                    pltpu.VMEM((CHL // 2, 2 * D), F32)]
    scratch += [
        pltpu.VMEM((2 * D,), F32),
        pltpu.VMEM_SHARED((N_PAD, 2 * D), F32),
    ] + [pltpu.SemaphoreType.DMA] * 10
    return pl.kernel(
        body, out_type=tuple(out_type),
        mesh=plsc.VectorSubcoreMesh(**_SC_MESH),
        scratch_types=tuple(scratch))


def _dot(a, b):
    return jnp.dot(a, b, preferred_element_type=F32)


def _tab0_body(emb_ref, w1_ref, b1_ref, w2_ref, b2_ref, wetop_ref, webot_ref,
               se_ref, de_ref, a_ref, ab_ref, b_ref, bb_ref, d_ref, db_ref,
               e_ref, eb_ref,
               p1_o, p2_o, l1_o, l2_o, tb_o, td_o, te_o, nt_o):
    emb = emb_ref[...]
    p1_o[...] = _dot(emb, w1_ref[...]) + b1_ref[...]
    p2_o[...] = _dot(emb, w2_ref[...]) + b2_ref[...]
    l1_o[...] = _dot(se_ref[...], wetop_ref[...])
    l2_o[...] = _dot(de_ref[...], webot_ref[...])
    tb_o[...] = _dot(emb, b_ref[...]) + bb_ref[...]
    td_o[...] = _dot(emb, d_ref[...]) + db_ref[...]
    te_o[...] = _dot(emb, e_ref[...]) + eb_ref[...]
    ta = _dot(emb, a_ref[...]) + ab_ref[...]
    nt_o[...] = jnp.concatenate([emb, ta], axis=1)


def _tab0(emb_h, mp, lp0):
    full = pl.BlockSpec(None, lambda: tuple())
    outs = (
        jax.ShapeDtypeStruct((V, D * D), F32),
        jax.ShapeDtypeStruct((V, D), F32),
        jax.ShapeDtypeStruct((V, D), F32),
        jax.ShapeDtypeStruct((V, D), F32),
        jax.ShapeDtypeStruct((V, D), F32),
        jax.ShapeDtypeStruct((V, D), F32),
        jax.ShapeDtypeStruct((V, D), F32),
        jax.ShapeDtypeStruct((V, 2 * D), F32),
    )
    args = (emb_h, mp['W1'], mp['b1'].reshape(1, -1), mp['W2'],
            mp['b2'].reshape(1, -1), mp['W_ep'][:D], mp['W_ep'][D:],
            mp['src_emb'], mp['dst_emb'],
            lp0['A'], lp0['Ab'].reshape(1, -1), lp0['B'],
            lp0['Bb'].reshape(1, -1), lp0['D'], lp0['Db'].reshape(1, -1),
            lp0['E'], lp0['Eb'].reshape(1, -1))
    return pl.pallas_call(
        _tab0_body,
        out_shape=outs,
    )(*args)


_CB = 2000


def _count_body(hs_ref, hd_ref, cnt_o, acc):
    i = pl.program_id(0)

    @pl.when(i == 0)
    def _():
        acc[...] = jnp.zeros_like(acc)

    lanes = lax.broadcasted_iota(jnp.int32, (_CB, V), 1)
    ohs = (hs_ref[...] == lanes).astype(jnp.bfloat16)
    ohd = (hd_ref[...] == lanes).astype(jnp.bfloat16)
    acc[...] += lax.dot_general(ohs, ohd, (((0,), (0,)), ((), ())),
                                preferred_element_type=F32)

    @pl.when(i == pl.num_programs(0) - 1)
    def _():
        cnt_o[...] = acc[...]


def _count(hs_col, hd_col):
    grid = E // _CB
    return pl.pallas_call(
        _count_body,
        grid=(grid,),
        in_specs=[pl.BlockSpec((_CB, 1), lambda i: (i, 0)),
                  pl.BlockSpec((_CB, 1), lambda i: (i, 0))],
        out_specs=pl.BlockSpec((V, V), lambda i: (0, 0)),
        out_shape=jax.ShapeDtypeStruct((V, V), F32),
        scratch_shapes=[pltpu.VMEM((V, V), F32)],
    )(hs_col, hd_col)


def _tab1_body(p1_ref, p2_ref, w3_ref, b3_ref, l1_ref, l2_ref, bep_ref,
               cnt_ref, tx_o, s1_o, s2_o, s1a, s2a):
    u = pl.program_id(0)

    @pl.when(u == 0)
    def _():
        s1a[...] = jnp.zeros_like(s1a)
        s2a[...] = jnp.zeros_like(s2a)

    p1u = p1_ref[0]
    tu = lax.dot_general(p2_ref[...], p1u, (((1,), (1,)), ((), ())),
                         preferred_element_type=F32)
    tx = (_dot(tu, w3_ref[...]) + b3_ref[...] + l1_ref[0]
          + l2_ref[...] + bep_ref[...])
    tx_o[...] = tx
    w = cnt_ref[0] * (1.0 / E)
    s1a[...] += _dot(w, tx)
    s2a[...] += _dot(w, tx * tx)

    @pl.when(u == pl.num_programs(0) - 1)
    def _():
        s1_o[...] = s1a[...]
        s2_o[...] = s2a[...]


def _tab1(p1v3, p2v, mp, l1t, l2t, cnt):
    return pl.pallas_call(
        _tab1_body,
        grid=(V,),
        in_specs=[
            pl.BlockSpec((1, D, D), lambda u: (u, 0, 0)),
            pl.BlockSpec((V, D), lambda u: (0, 0)),
            pl.BlockSpec((D, D), lambda u: (0, 0)),
            pl.BlockSpec((1, D), lambda u: (0, 0)),
            pl.BlockSpec((1, 1, D), lambda u: (u, 0, 0)),
            pl.BlockSpec((V, D), lambda u: (0, 0)),
            pl.BlockSpec((1, D), lambda u: (0, 0)),
            pl.BlockSpec((1, 1, V), lambda u: (u, 0, 0)),
        ],
        out_specs=[
            pl.BlockSpec((V, D), lambda u: (u, 0)),
            pl.BlockSpec((1, D), lambda u: (0, 0)),
            pl.BlockSpec((1, D), lambda u: (0, 0)),
        ],
        out_shape=[
            jax.ShapeDtypeStruct((V * V, D), F32),
            jax.ShapeDtypeStruct((1, D), F32),
            jax.ShapeDtypeStruct((1, D), F32),
        ],
        scratch_shapes=[pltpu.VMEM((1, D), F32), pltpu.VMEM((1, D), F32)],
    )(p1v3, p2v, mp['W_ep3'], mp['b_ep3'].reshape(1, -1),
      l1t.reshape(V, 1, D), l2t, mp['b_ep'].reshape(1, -1),
      cnt.reshape(V, 1, V))


def _tab2_body(tx_ref, s1_ref, s2_ref, g_ref, b_ref, td_ref, te_ref, tb_ref,
               c_ref, cb_ref, cnt_ref,
               tee_o, teh_o, tsg_o, s1e_o, s2e_o, s1a, s2a):
    u = pl.program_id(0)

    @pl.when(u == 0)
    def _():
        s1a[...] = jnp.zeros_like(s1a)
        s2a[...] = jnp.zeros_like(s2a)

    mu = s1_ref[...]
    inv = lax.rsqrt(s2_ref[...] - mu * mu + EPS)
    tee = jnp.maximum((tx_ref[...] - mu) * inv * g_ref[...] + b_ref[...], 0.0)
    tee_o[...] = tee
    teh = _dot(tee, c_ref[...]) + cb_ref[...] + td_ref[...] + te_ref[0]
    teh_o[...] = teh
    tsig = jax.nn.sigmoid(teh)
    tsg_o[...] = jnp.concatenate([tsig, tsig * tb_ref[0]], axis=1)
    w = cnt_ref[0] * (1.0 / E)
    s1a[...] += _dot(w, teh)
    s2a[...] += _dot(w, teh * teh)

    @pl.when(u == pl.num_programs(0) - 1)
    def _():
        s1e_o[...] = s1a[...]
        s2e_o[...] = s2a[...]


def _tab2(tx, s1m, s2m, mp, lp0, tbt, tdt, tet, cnt):
    return pl.pallas_call(
        _tab2_body,
        grid=(V,),
        in_specs=[
            pl.BlockSpec((V, D), lambda u: (u, 0)),
            pl.BlockSpec((1, D), lambda u: (0, 0)),
            pl.BlockSpec((1, D), lambda u: (0, 0)),
            pl.BlockSpec((1, D), lambda u: (0, 0)),
            pl.BlockSpec((1, D), lambda u: (0, 0)),
            pl.BlockSpec((V, D), lambda u: (0, 0)),
            pl.BlockSpec((1, 1, D), lambda u: (u, 0, 0)),
            pl.BlockSpec((1, 1, D), lambda u: (u, 0, 0)),
            pl.BlockSpec((D, D), lambda u: (0, 0)),
            pl.BlockSpec((1, D), lambda u: (0, 0)),
            pl.BlockSpec((1, 1, V), lambda u: (u, 0, 0)),
        ],
        out_specs=[
            pl.BlockSpec((V, D), lambda u: (u, 0)),
            pl.BlockSpec((V, D), lambda u: (u, 0)),
            pl.BlockSpec((V, 2 * D), lambda u: (u, 0)),
            pl.BlockSpec((1, D), lambda u: (0, 0)),
            pl.BlockSpec((1, D), lambda u: (0, 0)),
        ],
        out_shape=[
            jax.ShapeDtypeStruct((V * V, D), F32),
            jax.ShapeDtypeStruct((V * V, D), F32),
            jax.ShapeDtypeStruct((V * V, 2 * D), F32),
            jax.ShapeDtypeStruct((1, D), F32),
            jax.ShapeDtypeStruct((1, D), F32),
        ],
        scratch_shapes=[pltpu.VMEM((1, D), F32), pltpu.VMEM((1, D), F32)],
    )(tx, s1m, s2m, mp['bn_g'].reshape(1, -1), mp['bn_b'].reshape(1, -1),
      tdt, tet.reshape(V, 1, D), tbt.reshape(V, 1, D), lp0['C'],
      lp0['Cb'].reshape(1, -1), cnt.reshape(V, 1, V))


def _tab3_body(tee_ref, teh_ref, s1_ref, s2_ref, g_ref, b_ref, c_ref, cb_ref,
               tq_o):
    mu = s1_ref[...]
    inv = lax.rsqrt(s2_ref[...] - mu * mu + EPS)
    te2 = tee_ref[...] + jnp.maximum(
        (teh_ref[...] - mu) * inv * g_ref[...] + b_ref[...], 0.0)
    tq_o[...] = jnp.concatenate(
        [_dot(te2, c_ref[...]) + cb_ref[...], te2], axis=1)


def _tab3(tee, teh, s1e, s2e, lp0, lp1):
    return pl.pallas_call(
        _tab3_body,
        grid=(V,),
        in_specs=[
            pl.BlockSpec((V, D), lambda u: (u, 0)),
            pl.BlockSpec((V, D), lambda u: (u, 0)),
            pl.BlockSpec((1, D), lambda u: (0, 0)),
            pl.BlockSpec((1, D), lambda u: (0, 0)),
            pl.BlockSpec((1, D), lambda u: (0, 0)),
            pl.BlockSpec((1, D), lambda u: (0, 0)),
            pl.BlockSpec((D, D), lambda u: (0, 0)),
            pl.BlockSpec((1, D), lambda u: (0, 0)),
        ],
        out_specs=pl.BlockSpec((V, 2 * D), lambda u: (u, 0)),
        out_shape=jax.ShapeDtypeStruct((V * V, 2 * D), F32),
    )(tee, teh, s1e, s2e, lp0['bne_g'].reshape(1, -1),
      lp0['bne_b'].reshape(1, -1), lp1['C'], lp1['Cb'].reshape(1, -1))


_NB = 2048


def _row_mask(nrows, limit):
    rows = (pl.program_id(0) * nrows
            + lax.broadcasted_iota(jnp.int32, (nrows, 1), 0))
    return rows < limit


def _mm_body(hh_ref, a_ref, ab_ref, b_ref, bb_ref, d_ref, db_ref,
             e_ref, eb_ref, ah_o, dh_o, ebo_o):
    m = _row_mask(_NB, N)
    hh = hh_ref[...]
    ah_o[...] = jnp.where(m, _dot(hh, a_ref[...]) + ab_ref[...], 0.0)
    dh = jnp.where(m, _dot(hh, d_ref[...]) + db_ref[...], 0.0)
    dh_o[...] = jnp.concatenate([dh, jnp.zeros_like(dh)], axis=1)
    eh = _dot(hh, e_ref[...]) + eb_ref[...]
    bh = _dot(hh, b_ref[...]) + bb_ref[...]
    ebo_o[...] = jnp.where(m, jnp.concatenate([eh, bh], axis=1), 0.0)


def _tc_mm(hh, lp):
    return pl.pallas_call(
        _mm_body,
        grid=(N_PAD // _NB,),
        in_specs=[pl.BlockSpec((_NB, D), lambda i: (i, 0))]
        + [pl.BlockSpec((D, D), lambda i: (0, 0)),
           pl.BlockSpec((1, D), lambda i: (0, 0))] * 4,
        out_specs=[
            pl.BlockSpec((_NB, D), lambda i: (i, 0)),
            pl.BlockSpec((_NB, 2 * D), lambda i: (i, 0)),
            pl.BlockSpec((_NB, 2 * D), lambda i: (i, 0)),
        ],
        out_shape=[
            jax.ShapeDtypeStruct((N_PAD, D), F32),
            jax.ShapeDtypeStruct((N_PAD, 2 * D), F32),
            jax.ShapeDtypeStruct((N_PAD, 2 * D), F32),
        ],
    )(hh, lp['A'], lp['Ab'].reshape(1, -1), lp['B'], lp['Bb'].reshape(1, -1),
      lp['D'], lp['Db'].reshape(1, -1), lp['E'], lp['Eb'].reshape(1, -1))


def _ha_body(ah_ref, dn_ref, hn_o, s1_o, s2_o, s1a, s2a):
    i = pl.program_id(0)

    @pl.when(i == 0)
    def _():
        s1a[...] = jnp.zeros_like(s1a)
        s2a[...] = jnp.zeros_like(s2a)

    dn = dn_ref[0] + dn_ref[1]
    den = dn[:, :D]
    num = dn[:, D:]
    hn = ah_ref[...] + num / (den + 1e-6)
    hn = jnp.where(_row_mask(_NB, N), hn, 0.0)
    hn_o[...] = hn
    s1a[...] += jnp.sum(hn, axis=0, keepdims=True)
    s2a[...] += jnp.sum(hn * hn, axis=0, keepdims=True)

    @pl.when(i == pl.num_programs(0) - 1)
    def _():
        s1_o[...] = s1a[...]
        s2_o[...] = s2a[...]


def _tc_ha(ah, dn_p):
    return pl.pallas_call(
        _ha_body,
        grid=(N_PAD // _NB,),
        in_specs=[
            pl.BlockSpec((_NB, D), lambda i: (i, 0)),
            pl.BlockSpec((2, _NB, 2 * D), lambda i: (0, i, 0)),
        ],
        out_specs=[
            pl.BlockSpec((_NB, D), lambda i: (i, 0)),
            pl.BlockSpec((1, D), lambda i: (0, 0)),
            pl.BlockSpec((1, D), lambda i: (0, 0)),
        ],
        out_shape=[
            jax.ShapeDtypeStruct((N_PAD, D), F32),
            jax.ShapeDtypeStruct((1, D), F32),
            jax.ShapeDtypeStruct((1, D), F32),
        ],
        scratch_shapes=[pltpu.VMEM((1, D), F32), pltpu.VMEM((1, D), F32)],
    )(ah, dn_p)


def _hb_body(hh_ref, hn_ref, s1_ref, s2_ref, g_ref, b_ref, hho_o):
    mu = s1_ref[...] * (1.0 / N)
    var = s2_ref[...] * (1.0 / N) - mu * mu
    inv = lax.rsqrt(var + EPS)
    hho = hh_ref[...] + jnp.maximum(
        (hn_ref[...] - mu) * inv * g_ref[...] + b_ref[...], 0.0)
    hho_o[...] = jnp.where(_row_mask(_NB, N), hho, 0.0)


def _tc_hb(hh, hn, s1, s2, lp):
    return pl.pallas_call(
        _hb_body,
        grid=(N_PAD // _NB,),
        in_specs=[
            pl.BlockSpec((_NB, D), lambda i: (i, 0)),
            pl.BlockSpec((_NB, D), lambda i: (i, 0)),
            pl.BlockSpec((1, D), lambda i: (0, 0)),
            pl.BlockSpec((1, D), lambda i: (0, 0)),
            pl.BlockSpec((1, D), lambda i: (0, 0)),
            pl.BlockSpec((1, D), lambda i: (0, 0)),
        ],
        out_specs=pl.BlockSpec((_NB, D), lambda i: (i, 0)),
        out_shape=jax.ShapeDtypeStruct((N_PAD, D), F32),
    )(hh, hn, s1, s2, lp['bnh_g'].reshape(1, -1), lp['bnh_b'].reshape(1, -1))


_EB = 2048


def _make_e_body(with_ce):
    def body(*refs):
        if with_ce:
            (eh_ref, ee_ref, ss_ref, g_ref, b_ref, c_ref, cb_ref,
             eo_o, ce_o) = refs
        else:
            (eh_ref, ee_ref, ss_ref, g_ref, b_ref, eo_o) = refs
        ss = jnp.sum(ss_ref[...], axis=0, keepdims=True) * (1.0 / E)
        mu = ss[:, :D]
        inv = lax.rsqrt(ss[:, D:] - mu * mu + EPS)
        mu2 = jnp.concatenate([mu, mu], axis=1)
        inv2 = jnp.concatenate([inv, inv], axis=1)
        g2 = jnp.concatenate([g_ref[...], g_ref[...]], axis=1)
        b2 = jnp.concatenate([b_ref[...], b_ref[...]], axis=1)
        eo = ee_ref[...] + jnp.maximum(
            (eh_ref[...] - mu2) * inv2 * g2 + b2, 0.0)
        m = _row_mask(_EB, E // 2)
        eo = jnp.where(m, eo, 0.0)
        eo_o[...] = eo
        if with_ce:
            ce_o[...] = _dot(eo, c_ref[...]) + cb_ref[...]
    return body


def _tc_e(ehat, eein, ss, lp, lp_next):
    with_ce = lp_next is not None
    in_specs = [
        pl.BlockSpec((_EB, 2 * D), lambda i: (i, 0)),
        pl.BlockSpec((_EB, 2 * D), lambda i: (i, 0)),
        pl.BlockSpec((NW, 2 * D), lambda i: (0, 0)),
        pl.BlockSpec((1, D), lambda i: (0, 0)),
        pl.BlockSpec((1, D), lambda i: (0, 0)),
    ]
    args = [ehat, eein, ss, lp['bne_g'].reshape(1, -1),
            lp['bne_b'].reshape(1, -1)]
    out_specs = [pl.BlockSpec((_EB, 2 * D), lambda i: (i, 0))]
    out_shape = [jax.ShapeDtypeStruct((E_HALF, 2 * D), F32)]
    if with_ce:
        zdd = jnp.zeros((D, D), F32)
        c2 = jnp.block([[lp_next['C'], zdd], [zdd, lp_next['C']]])
        cb2 = jnp.tile(lp_next['Cb'], 2).reshape(1, -1)
        in_specs += [pl.BlockSpec((2 * D, 2 * D), lambda i: (0, 0)),
                     pl.BlockSpec((1, 2 * D), lambda i: (0, 0))]
        args += [c2, cb2]
        out_specs += [pl.BlockSpec((_EB, 2 * D), lambda i: (i, 0))]
        out_shape += [jax.ShapeDtypeStruct((E_HALF, 2 * D), F32)]
    res = pl.pallas_call(
        _make_e_body(with_ce),
        grid=(E_HALF // _EB,),
        in_specs=in_specs,
        out_specs=out_specs,
        out_shape=out_shape,
    )(*args)
    return res if with_ce else (res[0], None)


def _mlp_body(hh_ref, w0_ref, b0_ref, w1_ref, b1_ref, w2_ref, b2_ref, y_o):
    y = jnp.maximum(_dot(hh_ref[...], w0_ref[...]) + b0_ref[...], 0.0)
    y = jnp.maximum(_dot(y, w1_ref[...]) + b1_ref[...], 0.0)
    y_o[...] = _dot(y, w2_ref[...]) + b2_ref[...]


def _tc_mlp(hh, mlp):
    d0 = mlp['W0'].shape[1]
    d1 = mlp['W1'].shape[1]
    d2 = mlp['W2'].shape[1]
    return pl.pallas_call(
        _mlp_body,
        grid=(N_PAD // _NB,),
        in_specs=[
            pl.BlockSpec((_NB, D), lambda i: (i, 0)),
            pl.BlockSpec((D, d0), lambda i: (0, 0)),
            pl.BlockSpec((1, d0), lambda i: (0, 0)),
            pl.BlockSpec((d0, d1), lambda i: (0, 0)),
            pl.BlockSpec((1, d1), lambda i: (0, 0)),
            pl.BlockSpec((d1, d2), lambda i: (0, 0)),
            pl.BlockSpec((1, d2), lambda i: (0, 0)),
        ],
        out_specs=pl.BlockSpec((_NB, d2), lambda i: (i, 0)),
        out_shape=jax.ShapeDtypeStruct((N_PAD, d2), F32),
    )(hh, mlp['W0'], mlp['b0'].reshape(1, -1), mlp['W1'],
      mlp['b1'].reshape(1, -1), mlp['W2'], mlp['b2'].reshape(1, -1))


def kernel(h, edge_index, e, emb_h, gtp, merg, layers, mlp):
    del e, gtp
    h = h.astype(jnp.int32)
    src = edge_index[0].astype(jnp.int32)
    dst = edge_index[1].astype(jnp.int32)

    h_pad = jnp.concatenate([h, jnp.zeros((N_PAD - N,), jnp.int32)])
    pad_e = jnp.full((E_PAD - E,), DUMMY, jnp.int32)
    src2 = jnp.concatenate([src, pad_e]).reshape(NW, EPW)
    dst2 = jnp.concatenate([dst, pad_e]).reshape(NW, EPW)
    zrow = jnp.zeros((TPS, 2 * D), F32)

    lp = layers

    p1v, p2v, l1t, l2t, tbt, tdt, tet, nt = _tab0(emb_h, merg, lp[0])
    src3 = src2.reshape(NW, NCH, CH)
    dst3 = dst2.reshape(NW, NCH, CH)
    pid2, hs2, hd2, ntg = _sc0(h_pad, src3, dst3, nt)
    pid3 = pid2.reshape(NW, NCH, CH)
    hs_col = hs2.reshape(E_PAD, 1)[:E]
    hd_col = hd2.reshape(E_PAD, 1)[:E]

    cnt = _count(hs_col, hd_col)
    tx, s1m, s2m = _tab1(p1v.reshape(V, D, D), p2v, merg, l1t, l2t, cnt)
    tee, teh, tsg, s1e1, s2e1 = _tab2(
        tx, s1m, s2m, merg, lp[0], tbt, tdt, tet, cnt)
    tq = _tab3(tee, teh, s1e1, s2e1, lp[0], lp[1])

    dn_p = _sc_l1(pid3, dst3, tsg, zrow)
    hemb = ntg[:, :D]
    ah1 = ntg[:, D:]
    hn, s1h, s2h = _tc_ha(ah1, dn_p)
    hh = _tc_hb(hemb, hn, s1h, s2h, lp[0])

    sc_l2 = _make_sc_layer(gather_ce=True, with_stats=True)
    sc_l3 = _make_sc_layer(gather_ce=False, with_stats=True)
    sc_l4 = _make_sc_layer(gather_ce=False, with_stats=False)

    ah, dh, ebt = _tc_mm(hh, lp[1])
    dn_p, ehat, ss, eein = sc_l2(src3, dst3, pid3, tq, dh, ebt, zrow)
    hn, s1h, s2h = _tc_ha(ah, dn_p)
    hh = _tc_hb(hh, hn, s1h, s2h, lp[1])
    ee, ce = _tc_e(ehat, eein, ss, lp[1], lp[2])

    ah, dh, ebt = _tc_mm(hh, lp[2])
    dn_p, ehat, ss = sc_l3(src3, dst3, ce, dh, ebt, zrow)
    hn, s1h, s2h = _tc_ha(ah, dn_p)
    hh = _tc_hb(hh, hn, s1h, s2h, lp[2])
    ee, ce = _tc_e(ehat, ee, ss, lp[2], lp[3])

    ah, dh, ebt = _tc_mm(hh, lp[3])
    dn_p, = sc_l4(src3, dst3, ce, dh, ebt, zrow)
    hn, s1h, s2h = _tc_ha(ah, dn_p)
    hh = _tc_hb(hh, hn, s1h, s2h, lp[3])

    return _tc_mlp(hh, mlp)[:N]
